# Initial kernel scaffold; baseline (speedup 1.0000x reference)
import functools

import jax
import jax.numpy as jnp
from jax import lax
from jax.experimental import pallas as pl
from jax.experimental.pallas import tpu as pltpu
from jax.experimental.pallas import tpu_sc as plsc

N = 100000
E = 1600000
D = 16
NP = 100008
CHUNK = 128
EPAD = 1638400
ROWS2D = EPAD // CHUNK
KB = 16
NB = 25
ZROWS = 6256
ZCAP = N - ZROWS
BM = 1000


def _agg_edges(h_hbm, src_hbm, dst_hbm, acc_sh, src_v, dst_v, rows_v, sem,
               cnt_sh=None, ones_v=None):
    c = lax.axis_index("c")
    s = lax.axis_index("s")
    tile_row0 = (c * 16 + s) * (NB * KB)

    def body(b, carry):
        row0 = tile_row0 + b * KB
        pltpu.sync_copy(src_hbm.at[pl.ds(row0, KB)], src_v)
        pltpu.sync_copy(dst_hbm.at[pl.ds(row0, KB)], dst_v)
        for j in range(KB):
            pltpu.async_copy(h_hbm.at[src_v.at[j]], rows_v, sem).wait()
            pltpu.sync_copy(rows_v, acc_sh.at[dst_v.at[j]], add=True)
            if cnt_sh is not None:
                pltpu.sync_copy(ones_v, cnt_sh.at[dst_v.at[j]], add=True)
        return carry

    lax.fori_loop(0, NB, body, 0)


@functools.lru_cache(maxsize=None)
def _sc_agg_count():
    mesh = plsc.VectorSubcoreMesh(core_axis_name="c", subcore_axis_name="s")

    @functools.partial(
        pl.kernel,
        mesh=mesh,
        out_type=(
            jax.ShapeDtypeStruct((2, N, D), jnp.float32),
            jax.ShapeDtypeStruct((2, N), jnp.float32),
        ),
        scratch_types=[
            pltpu.VMEM((KB, CHUNK), jnp.int32),
            pltpu.VMEM((KB, CHUNK), jnp.int32),
            pltpu.VMEM((CHUNK, D), jnp.float32),
            pltpu.VMEM((CHUNK,), jnp.float32),
            pltpu.VMEM_SHARED((NP, D), jnp.float32),
            pltpu.VMEM_SHARED((NP,), jnp.float32),
            pltpu.SemaphoreType.DMA,
        ],
    )
    def k(h_hbm, src_hbm, dst_hbm, z2d_hbm, z1d_hbm, p_hbm, cnt_hbm,
          src_v, dst_v, rows_v, ones_v, acc_sh, cnt_sh, sem):
        c = lax.axis_index("c")
        s = lax.axis_index("s")
        o = jnp.minimum(s * ZROWS, ZCAP)
        pltpu.sync_copy(z2d_hbm, acc_sh.at[pl.ds(o, ZROWS)])
        pltpu.sync_copy(z1d_hbm, cnt_sh.at[pl.ds(o, ZROWS)])
        for i in range(CHUNK // 16):
            ones_v[pl.ds(i * 16, 16)] = jnp.ones((16,), jnp.float32)
        plsc.subcore_barrier()
        _agg_edges(h_hbm, src_hbm, dst_hbm, acc_sh, src_v, dst_v, rows_v, sem,
                   cnt_sh=cnt_sh, ones_v=ones_v)
        plsc.subcore_barrier()
        pltpu.sync_copy(acc_sh.at[pl.ds(o, ZROWS)], p_hbm.at[c, pl.ds(o, ZROWS)])
        pltpu.sync_copy(cnt_sh.at[pl.ds(o, ZROWS)], cnt_hbm.at[c, pl.ds(o, ZROWS)])

    return k


@functools.lru_cache(maxsize=None)
def _sc_agg():
    mesh = plsc.VectorSubcoreMesh(core_axis_name="c", subcore_axis_name="s")

    @functools.partial(
        pl.kernel,
        mesh=mesh,
        out_type=jax.ShapeDtypeStruct((2, N, D), jnp.float32),
        scratch_types=[
            pltpu.VMEM((KB, CHUNK), jnp.int32),
            pltpu.VMEM((KB, CHUNK), jnp.int32),
            pltpu.VMEM((CHUNK, D), jnp.float32),
            pltpu.VMEM_SHARED((NP, D), jnp.float32),
            pltpu.SemaphoreType.DMA,
        ],
    )
    def k(h_hbm, src_hbm, dst_hbm, z2d_hbm, p_hbm,
          src_v, dst_v, rows_v, acc_sh, sem):
        c = lax.axis_index("c")
        s = lax.axis_index("s")
        o = jnp.minimum(s * ZROWS, ZCAP)
        pltpu.sync_copy(z2d_hbm, acc_sh.at[pl.ds(o, ZROWS)])
        plsc.subcore_barrier()
        _agg_edges(h_hbm, src_hbm, dst_hbm, acc_sh, src_v, dst_v, rows_v, sem)
        plsc.subcore_barrier()
        pltpu.sync_copy(acc_sh.at[pl.ds(o, ZROWS)], p_hbm.at[c, pl.ds(o, ZROWS)])

    return k


def _fc1_body(x_ref, w_ref, b_ref, o_ref):
    o_ref[...] = jnp.maximum(
        jnp.dot(x_ref[...], w_ref[...], preferred_element_type=jnp.float32)
        + b_ref[...], 0.0)


def _tc_fc1(x, W, b):
    return pl.pallas_call(
        _fc1_body,
        grid=(N // BM,),
        in_specs=[
            pl.BlockSpec((BM, 128), lambda i: (i, 0)),
            pl.BlockSpec((128, D), lambda i: (0, 0)),
            pl.BlockSpec((1, D), lambda i: (0, 0)),
        ],
        out_specs=pl.BlockSpec((BM, D), lambda i: (i, 0)),
        out_shape=jax.ShapeDtypeStruct((N, D), jnp.float32),
    )(x, W, b.reshape(1, D))


def _mid_body(h1_ref, p_ref, cnt_ref, wl_ref, wr_ref, b1_ref, w2l_ref, w2r_ref,
              y_ref, z_ref):
    inv = 1.0 / jnp.maximum(cnt_ref[0] + cnt_ref[1], 1.0)
    a1 = (p_ref[0] + p_ref[1]) * inv[:, None]
    h2 = jnp.maximum(
        jnp.dot(a1, wl_ref[...], preferred_element_type=jnp.float32)
        + jnp.dot(h1_ref[...], wr_ref[...], preferred_element_type=jnp.float32)
        + b1_ref[...], 0.0)
    y_ref[...] = jnp.dot(h2, w2l_ref[...], preferred_element_type=jnp.float32)
    z_ref[...] = jnp.dot(h2, w2r_ref[...], preferred_element_type=jnp.float32)


def _tc_mid(h1, p, cnt, c1_Wl, c1_Wr, c1_b, c2_Wl, c2_Wr):
    return pl.pallas_call(
        _mid_body,
        grid=(N // BM,),
        in_specs=[
            pl.BlockSpec((BM, D), lambda i: (i, 0)),
            pl.BlockSpec((2, BM, D), lambda i: (0, i, 0)),
            pl.BlockSpec((2, BM), lambda i: (0, i)),
            pl.BlockSpec((D, 32), lambda i: (0, 0)),
            pl.BlockSpec((D, 32), lambda i: (0, 0)),
            pl.BlockSpec((1, 32), lambda i: (0, 0)),
            pl.BlockSpec((32, D), lambda i: (0, 0)),
            pl.BlockSpec((32, D), lambda i: (0, 0)),
        ],
        out_specs=[
            pl.BlockSpec((BM, D), lambda i: (i, 0)),
            pl.BlockSpec((BM, D), lambda i: (i, 0)),
        ],
        out_shape=[
            jax.ShapeDtypeStruct((N, D), jnp.float32),
            jax.ShapeDtypeStruct((N, D), jnp.float32),
        ],
    )(h1, p, cnt, c1_Wl, c1_Wr, c1_b.reshape(1, 32), c2_Wl, c2_Wr)


def _fin_body(q_ref, cnt_ref, z_ref, b2_ref, w_ref, b_ref, o_ref):
    inv = 1.0 / jnp.maximum(cnt_ref[0] + cnt_ref[1], 1.0)
    a2 = (q_ref[0] + q_ref[1]) * inv[:, None]
    h3 = jnp.maximum(a2 + z_ref[...] + b2_ref[...], 0.0)
    o = jnp.dot(h3, w_ref[...], preferred_element_type=jnp.float32) + b_ref[...]
    g = jax.nn.sigmoid(o[:, 1:2])
    fsi = jnp.maximum(o[:, 0:1], 0.0) + g
    mxi = jax.nn.sigmoid(o[:, 2:3])
    o_ref[...] = jnp.concatenate([fsi, g, mxi], axis=1)


def _tc_fin(q, cnt, z, c2_b, fc2_W, fc2_b):
    return pl.pallas_call(
        _fin_body,
        grid=(N // BM,),
        in_specs=[
            pl.BlockSpec((2, BM, D), lambda i: (0, i, 0)),
            pl.BlockSpec((2, BM), lambda i: (0, i)),
            pl.BlockSpec((BM, D), lambda i: (i, 0)),
            pl.BlockSpec((1, D), lambda i: (0, 0)),
            pl.BlockSpec((D, 3), lambda i: (0, 0)),
            pl.BlockSpec((1, 3), lambda i: (0, 0)),
        ],
        out_specs=pl.BlockSpec((BM, 3), lambda i: (i, 0)),
        out_shape=jax.ShapeDtypeStruct((N, 3), jnp.float32),
    )(q, cnt, z, c2_b.reshape(1, D), fc2_W, fc2_b.reshape(1, 3))


def kernel(x, edge_index, fc1_W, fc1_b, c1_Wl, c1_Wr, c1_b,
           c2_Wl, c2_Wr, c2_b, fc2_W, fc2_b):
    pad = EPAD - E
    src2d = jnp.concatenate(
        [edge_index[0], jnp.zeros((pad,), jnp.int32)]).reshape(ROWS2D, CHUNK)
    dst2d = jnp.concatenate(
        [edge_index[1], jnp.full((pad,), N, jnp.int32)]).reshape(ROWS2D, CHUNK)
    z2d = jnp.zeros((ZROWS, D), jnp.float32)
    z1d = jnp.zeros((ZROWS,), jnp.float32)

    h1 = _tc_fc1(x, fc1_W, fc1_b)
    p, cnt = _sc_agg_count()(h1, src2d, dst2d, z2d, z1d)
    y2, z2f = _tc_mid(h1, p, cnt, c1_Wl, c1_Wr, c1_b, c2_Wl, c2_Wr)
    q = _sc_agg()(y2, src2d, dst2d, z2d)
    return _tc_fin(q, cnt, z2f, c2_b, fc2_W, fc2_b)


# speedup vs baseline: 8.4871x; 8.4871x over previous
import functools

import jax
import jax.numpy as jnp
from jax import lax
from jax.experimental import pallas as pl
from jax.experimental.pallas import tpu as pltpu
from jax.experimental.pallas import tpu_sc as plsc

N = 100000
E = 1600000
D = 16
NP = 100008
CHUNK = 128
EPAD = 1638400
ROWS2D = EPAD // CHUNK
KB = 16
NB = 25
ZROWS = 6256
ZCAP = N - ZROWS
BM = 1000


def _agg_edges(h_hbm, src_hbm, dst_hbm, acc_sh, src_v, dst_v, rows_v, sem,
               cnt_sh=None, ones_v=None):
    c = lax.axis_index("c")
    s = lax.axis_index("s")
    tile_row0 = (c * 16 + s) * (NB * KB)

    def body(b, carry):
        row0 = tile_row0 + b * KB
        pltpu.sync_copy(src_hbm.at[pl.ds(row0, KB)], src_v)
        pltpu.sync_copy(dst_hbm.at[pl.ds(row0, KB)], dst_v)
        for j in range(KB):
            pltpu.async_copy(h_hbm.at[src_v.at[j]], rows_v, sem).wait()
            pltpu.sync_copy(rows_v, acc_sh.at[dst_v.at[j]], add=True)
            if cnt_sh is not None:
                pltpu.sync_copy(ones_v, cnt_sh.at[dst_v.at[j]], add=True)
        return carry

    lax.fori_loop(0, NB, body, 0)


ZB = 1024


def _fill_zeros_2d(zv):
    def fz(i, c):
        zv[i, :] = jnp.zeros((16,), jnp.float32)
        return c
    lax.fori_loop(0, ZB, fz, 0)


def _fill_zeros_1d(zv):
    def fz(i, c):
        zv[pl.ds(i * 16, 16)] = jnp.zeros((16,), jnp.float32)
        return c
    lax.fori_loop(0, ZB // 16, fz, 0)


def _zero_slice_2d(acc_sh, zv, o):
    for i in range(ZROWS // ZB):
        pltpu.sync_copy(zv, acc_sh.at[pl.ds(o + i * ZB, ZB)])
    rem = ZROWS % ZB
    pltpu.sync_copy(zv.at[pl.ds(0, rem)], acc_sh.at[pl.ds(o + ZROWS - rem, rem)])


def _zero_slice_1d(cnt_sh, zv, o):
    for i in range(ZROWS // ZB):
        pltpu.sync_copy(zv, cnt_sh.at[pl.ds(o + i * ZB, ZB)])
    rem = ZROWS % ZB
    pltpu.sync_copy(zv.at[pl.ds(0, rem)], cnt_sh.at[pl.ds(o + ZROWS - rem, rem)])


@functools.lru_cache(maxsize=None)
def _sc_agg_count():
    mesh = plsc.VectorSubcoreMesh(core_axis_name="c", subcore_axis_name="s")

    @functools.partial(
        pl.kernel,
        mesh=mesh,
        compiler_params=pltpu.CompilerParams(use_tc_tiling_on_sc=False),
        out_type=(
            jax.ShapeDtypeStruct((2, N, D), jnp.float32),
            jax.ShapeDtypeStruct((2 * N,), jnp.float32),
        ),
        scratch_types=[
            pltpu.VMEM((KB, CHUNK), jnp.int32),
            pltpu.VMEM((KB, CHUNK), jnp.int32),
            pltpu.VMEM((CHUNK, D), jnp.float32),
            pltpu.VMEM((CHUNK,), jnp.float32),
            pltpu.VMEM((ZB, D), jnp.float32),
            pltpu.VMEM((ZB,), jnp.float32),
            pltpu.VMEM_SHARED((NP, D), jnp.float32),
            pltpu.VMEM_SHARED((NP,), jnp.float32),
            pltpu.SemaphoreType.DMA,
        ],
    )
    def k(h_hbm, src_hbm, dst_hbm, p_hbm, cnt_hbm,
          src_v, dst_v, rows_v, ones_v, zv2d, zv1d, acc_sh, cnt_sh, sem):
        c = lax.axis_index("c")
        s = lax.axis_index("s")
        o = jnp.minimum(s * ZROWS, ZCAP)
        _fill_zeros_2d(zv2d)
        _fill_zeros_1d(zv1d)
        _zero_slice_2d(acc_sh, zv2d, o)
        _zero_slice_1d(cnt_sh, zv1d, o)
        for i in range(CHUNK // 16):
            ones_v[pl.ds(i * 16, 16)] = jnp.ones((16,), jnp.float32)
        plsc.subcore_barrier()
        _agg_edges(h_hbm, src_hbm, dst_hbm, acc_sh, src_v, dst_v, rows_v, sem,
                   cnt_sh=cnt_sh, ones_v=ones_v)
        plsc.subcore_barrier()
        pltpu.sync_copy(acc_sh.at[pl.ds(o, ZROWS)], p_hbm.at[c, pl.ds(o, ZROWS)])
        pltpu.sync_copy(cnt_sh.at[pl.ds(o, ZROWS)], cnt_hbm.at[pl.ds(c * N + o, ZROWS)])

    return k


@functools.lru_cache(maxsize=None)
def _sc_agg():
    mesh = plsc.VectorSubcoreMesh(core_axis_name="c", subcore_axis_name="s")

    @functools.partial(
        pl.kernel,
        mesh=mesh,
        compiler_params=pltpu.CompilerParams(use_tc_tiling_on_sc=False),
        out_type=jax.ShapeDtypeStruct((2, N, D), jnp.float32),
        scratch_types=[
            pltpu.VMEM((KB, CHUNK), jnp.int32),
            pltpu.VMEM((KB, CHUNK), jnp.int32),
            pltpu.VMEM((CHUNK, D), jnp.float32),
            pltpu.VMEM((ZB, D), jnp.float32),
            pltpu.VMEM_SHARED((NP, D), jnp.float32),
            pltpu.SemaphoreType.DMA,
        ],
    )
    def k(h_hbm, src_hbm, dst_hbm, p_hbm,
          src_v, dst_v, rows_v, zv2d, acc_sh, sem):
        c = lax.axis_index("c")
        s = lax.axis_index("s")
        o = jnp.minimum(s * ZROWS, ZCAP)
        _fill_zeros_2d(zv2d)
        _zero_slice_2d(acc_sh, zv2d, o)
        plsc.subcore_barrier()
        _agg_edges(h_hbm, src_hbm, dst_hbm, acc_sh, src_v, dst_v, rows_v, sem)
        plsc.subcore_barrier()
        pltpu.sync_copy(acc_sh.at[pl.ds(o, ZROWS)], p_hbm.at[c, pl.ds(o, ZROWS)])

    return k


def _fc1_body(x_ref, w_ref, b_ref, o_ref):
    o_ref[...] = jnp.maximum(
        jnp.dot(x_ref[...], w_ref[...], preferred_element_type=jnp.float32)
        + b_ref[...], 0.0)


def _tc_fc1(x, W, b):
    return pl.pallas_call(
        _fc1_body,
        grid=(N // BM,),
        in_specs=[
            pl.BlockSpec((BM, 128), lambda i: (i, 0)),
            pl.BlockSpec((128, D), lambda i: (0, 0)),
            pl.BlockSpec((1, D), lambda i: (0, 0)),
        ],
        out_specs=pl.BlockSpec((BM, D), lambda i: (i, 0)),
        out_shape=jax.ShapeDtypeStruct((N, D), jnp.float32),
    )(x, W, b.reshape(1, D))


def _mid_body(h1_ref, p_ref, cnt_ref, wl_ref, wr_ref, b1_ref, w2l_ref, w2r_ref,
              y_ref, z_ref):
    inv = 1.0 / jnp.maximum(cnt_ref[0] + cnt_ref[1], 1.0)
    a1 = (p_ref[0] + p_ref[1]) * inv
    h2 = jnp.maximum(
        jnp.dot(a1, wl_ref[...], preferred_element_type=jnp.float32)
        + jnp.dot(h1_ref[...], wr_ref[...], preferred_element_type=jnp.float32)
        + b1_ref[...], 0.0)
    y_ref[...] = jnp.dot(h2, w2l_ref[...], preferred_element_type=jnp.float32)
    z_ref[...] = jnp.dot(h2, w2r_ref[...], preferred_element_type=jnp.float32)


def _tc_mid(h1, p, cnt, c1_Wl, c1_Wr, c1_b, c2_Wl, c2_Wr):
    return pl.pallas_call(
        _mid_body,
        grid=(N // BM,),
        in_specs=[
            pl.BlockSpec((BM, D), lambda i: (i, 0)),
            pl.BlockSpec((2, BM, D), lambda i: (0, i, 0)),
            pl.BlockSpec((2, BM, 1), lambda i: (0, i, 0)),
            pl.BlockSpec((D, 32), lambda i: (0, 0)),
            pl.BlockSpec((D, 32), lambda i: (0, 0)),
            pl.BlockSpec((1, 32), lambda i: (0, 0)),
            pl.BlockSpec((32, D), lambda i: (0, 0)),
            pl.BlockSpec((32, D), lambda i: (0, 0)),
        ],
        out_specs=[
            pl.BlockSpec((BM, D), lambda i: (i, 0)),
            pl.BlockSpec((BM, D), lambda i: (i, 0)),
        ],
        out_shape=[
            jax.ShapeDtypeStruct((N, D), jnp.float32),
            jax.ShapeDtypeStruct((N, D), jnp.float32),
        ],
    )(h1, p, cnt.reshape(2, N, 1), c1_Wl, c1_Wr, c1_b.reshape(1, 32), c2_Wl, c2_Wr)


def _fin_body(q_ref, cnt_ref, z_ref, b2_ref, w_ref, b_ref, o_ref):
    inv = 1.0 / jnp.maximum(cnt_ref[0] + cnt_ref[1], 1.0)
    a2 = (q_ref[0] + q_ref[1]) * inv
    h3 = jnp.maximum(a2 + z_ref[...] + b2_ref[...], 0.0)
    o = jnp.dot(h3, w_ref[...], preferred_element_type=jnp.float32) + b_ref[...]
    g = jax.nn.sigmoid(o[:, 1:2])
    fsi = jnp.maximum(o[:, 0:1], 0.0) + g
    mxi = jax.nn.sigmoid(o[:, 2:3])
    o_ref[...] = jnp.concatenate([fsi, g, mxi], axis=1)


def _tc_fin(q, cnt, z, c2_b, fc2_W, fc2_b):
    return pl.pallas_call(
        _fin_body,
        grid=(N // BM,),
        in_specs=[
            pl.BlockSpec((2, BM, D), lambda i: (0, i, 0)),
            pl.BlockSpec((2, BM, 1), lambda i: (0, i, 0)),
            pl.BlockSpec((BM, D), lambda i: (i, 0)),
            pl.BlockSpec((1, D), lambda i: (0, 0)),
            pl.BlockSpec((D, 3), lambda i: (0, 0)),
            pl.BlockSpec((1, 3), lambda i: (0, 0)),
        ],
        out_specs=pl.BlockSpec((BM, 3), lambda i: (i, 0)),
        out_shape=jax.ShapeDtypeStruct((N, 3), jnp.float32),
    )(q, cnt.reshape(2, N, 1), z, c2_b.reshape(1, D), fc2_W, fc2_b.reshape(1, 3))


def kernel(x, edge_index, fc1_W, fc1_b, c1_Wl, c1_Wr, c1_b,
           c2_Wl, c2_Wr, c2_b, fc2_W, fc2_b):
    pad = EPAD - E
    src2d = jnp.concatenate(
        [edge_index[0], jnp.zeros((pad,), jnp.int32)]).reshape(ROWS2D, CHUNK)
    dst2d = jnp.concatenate(
        [edge_index[1], jnp.full((pad,), N, jnp.int32)]).reshape(ROWS2D, CHUNK)
    h1 = _tc_fc1(x, fc1_W, fc1_b)
    p, cnt = _sc_agg_count()(h1, src2d, dst2d)
    y2, z2f = _tc_mid(h1, p, cnt, c1_Wl, c1_Wr, c1_b, c2_Wl, c2_Wr)
    q = _sc_agg()(y2, src2d, dst2d)
    return _tc_fin(q, cnt, z2f, c2_b, fc2_W, fc2_b)


# speedup vs baseline: 9.8925x; 1.1656x over previous
import functools

import jax
import jax.numpy as jnp
from jax import lax
from jax.experimental import pallas as pl
from jax.experimental.pallas import tpu as pltpu
from jax.experimental.pallas import tpu_sc as plsc

N = 100000
E = 1600000
D = 16
NP = 100008
CHUNK = 128
EPAD = 1638400
ROWS2D = EPAD // CHUNK
KB = 4
NB = 100
ZROWS = 6256
ZCAP = N - ZROWS
BM = 1000


def _agg_edges(h_hbm, src_hbm, dst_hbm, acc_sh, src_v, dst_v, rows_v,
               gsem, ssem, cnt_sh=None, ones_v=None):
    c = lax.axis_index("c")
    s = lax.axis_index("s")
    tile_row0 = (c * 16 + s) * (NB * KB)

    def gather_descs(bb):
        return [
            pltpu.make_async_copy(
                h_hbm.at[src_v.at[bb, j]],
                rows_v.at[bb, pl.ds(j * CHUNK, CHUNK)],
                gsem.at[bb])
            for j in range(KB)
        ]

    def scatter_descs(bb):
        out = []
        for j in range(KB):
            out.append(pltpu.make_async_copy(
                rows_v.at[bb, pl.ds(j * CHUNK, CHUNK)],
                acc_sh.at[dst_v.at[bb, j]],
                ssem.at[bb]))
            if cnt_sh is not None:
                out.append(pltpu.make_async_copy(
                    ones_v, cnt_sh.at[dst_v.at[bb, j]], ssem.at[bb]))
        return out

    def body(b, carry):
        bb = b % 2
        row0 = tile_row0 + b * KB

        @pl.when(b >= 2)
        def _():
            for d in scatter_descs(bb):
                d.wait()

        pltpu.sync_copy(src_hbm.at[pl.ds(row0, KB)], src_v.at[bb])
        pltpu.sync_copy(dst_hbm.at[pl.ds(row0, KB)], dst_v.at[bb])
        for d in gather_descs(bb):
            d.start()
        for d in gather_descs(bb):
            d.wait()
        for d in scatter_descs(bb):
            d.start(add=True)
        return carry

    lax.fori_loop(0, NB, body, 0)
    for bb in (0, 1):
        for d in scatter_descs(bb):
            d.wait()


ZB = 256


def _fill_zeros_2d(zv):
    def fz(i, c):
        zv[i, :] = jnp.zeros((16,), jnp.float32)
        return c
    lax.fori_loop(0, ZB, fz, 0)


def _fill_zeros_1d(zv):
    def fz(i, c):
        zv[pl.ds(i * 16, 16)] = jnp.zeros((16,), jnp.float32)
        return c
    lax.fori_loop(0, ZB // 16, fz, 0)


def _zero_slice_2d(acc_sh, zv, o):
    for i in range(ZROWS // ZB):
        pltpu.sync_copy(zv, acc_sh.at[pl.ds(o + i * ZB, ZB)])
    rem = ZROWS % ZB
    pltpu.sync_copy(zv.at[pl.ds(0, rem)], acc_sh.at[pl.ds(o + ZROWS - rem, rem)])


def _zero_slice_1d(cnt_sh, zv, o):
    for i in range(ZROWS // ZB):
        pltpu.sync_copy(zv, cnt_sh.at[pl.ds(o + i * ZB, ZB)])
    rem = ZROWS % ZB
    pltpu.sync_copy(zv.at[pl.ds(0, rem)], cnt_sh.at[pl.ds(o + ZROWS - rem, rem)])


@functools.lru_cache(maxsize=None)
def _sc_agg_count():
    mesh = plsc.VectorSubcoreMesh(core_axis_name="c", subcore_axis_name="s")

    @functools.partial(
        pl.kernel,
        mesh=mesh,
        compiler_params=pltpu.CompilerParams(use_tc_tiling_on_sc=False),
        out_type=(
            jax.ShapeDtypeStruct((2, N, D), jnp.float32),
            jax.ShapeDtypeStruct((2 * N,), jnp.float32),
        ),
        scratch_types=[
            pltpu.VMEM((2, KB, CHUNK), jnp.int32),
            pltpu.VMEM((2, KB, CHUNK), jnp.int32),
            pltpu.VMEM((2, KB * CHUNK, D), jnp.float32),
            pltpu.VMEM((CHUNK,), jnp.float32),
            pltpu.VMEM((ZB, D), jnp.float32),
            pltpu.VMEM((ZB,), jnp.float32),
            pltpu.VMEM_SHARED((NP, D), jnp.float32),
            pltpu.VMEM_SHARED((NP,), jnp.float32),
            pltpu.SemaphoreType.DMA((2,)),
            pltpu.SemaphoreType.DMA((2,)),
        ],
    )
    def k(h_hbm, src_hbm, dst_hbm, p_hbm, cnt_hbm,
          src_v, dst_v, rows_v, ones_v, zv2d, zv1d, acc_sh, cnt_sh, gsem, ssem):
        c = lax.axis_index("c")
        s = lax.axis_index("s")
        o = jnp.minimum(s * ZROWS, ZCAP)
        _fill_zeros_2d(zv2d)
        _fill_zeros_1d(zv1d)
        _zero_slice_2d(acc_sh, zv2d, o)
        _zero_slice_1d(cnt_sh, zv1d, o)
        for i in range(CHUNK // 16):
            ones_v[pl.ds(i * 16, 16)] = jnp.ones((16,), jnp.float32)
        plsc.subcore_barrier()
        _agg_edges(h_hbm, src_hbm, dst_hbm, acc_sh, src_v, dst_v, rows_v,
                   gsem, ssem, cnt_sh=cnt_sh, ones_v=ones_v)
        plsc.subcore_barrier()
        pltpu.sync_copy(acc_sh.at[pl.ds(o, ZROWS)], p_hbm.at[c, pl.ds(o, ZROWS)])
        pltpu.sync_copy(cnt_sh.at[pl.ds(o, ZROWS)], cnt_hbm.at[pl.ds(c * N + o, ZROWS)])

    return k


@functools.lru_cache(maxsize=None)
def _sc_agg():
    mesh = plsc.VectorSubcoreMesh(core_axis_name="c", subcore_axis_name="s")

    @functools.partial(
        pl.kernel,
        mesh=mesh,
        compiler_params=pltpu.CompilerParams(use_tc_tiling_on_sc=False),
        out_type=jax.ShapeDtypeStruct((2, N, D), jnp.float32),
        scratch_types=[
            pltpu.VMEM((2, KB, CHUNK), jnp.int32),
            pltpu.VMEM((2, KB, CHUNK), jnp.int32),
            pltpu.VMEM((2, KB * CHUNK, D), jnp.float32),
            pltpu.VMEM((ZB, D), jnp.float32),
            pltpu.VMEM_SHARED((NP, D), jnp.float32),
            pltpu.SemaphoreType.DMA((2,)),
            pltpu.SemaphoreType.DMA((2,)),
        ],
    )
    def k(h_hbm, src_hbm, dst_hbm, p_hbm,
          src_v, dst_v, rows_v, zv2d, acc_sh, gsem, ssem):
        c = lax.axis_index("c")
        s = lax.axis_index("s")
        o = jnp.minimum(s * ZROWS, ZCAP)
        _fill_zeros_2d(zv2d)
        _zero_slice_2d(acc_sh, zv2d, o)
        plsc.subcore_barrier()
        _agg_edges(h_hbm, src_hbm, dst_hbm, acc_sh, src_v, dst_v, rows_v,
                   gsem, ssem)
        plsc.subcore_barrier()
        pltpu.sync_copy(acc_sh.at[pl.ds(o, ZROWS)], p_hbm.at[c, pl.ds(o, ZROWS)])

    return k


def _fc1_body(x_ref, w_ref, b_ref, o_ref):
    o_ref[...] = jnp.maximum(
        jnp.dot(x_ref[...], w_ref[...], preferred_element_type=jnp.float32)
        + b_ref[...], 0.0)


def _tc_fc1(x, W, b):
    return pl.pallas_call(
        _fc1_body,
        grid=(N // BM,),
        in_specs=[
            pl.BlockSpec((BM, 128), lambda i: (i, 0)),
            pl.BlockSpec((128, D), lambda i: (0, 0)),
            pl.BlockSpec((1, D), lambda i: (0, 0)),
        ],
        out_specs=pl.BlockSpec((BM, D), lambda i: (i, 0)),
        out_shape=jax.ShapeDtypeStruct((N, D), jnp.float32),
    )(x, W, b.reshape(1, D))


def _mid_body(h1_ref, p_ref, cnt_ref, wl_ref, wr_ref, b1_ref, w2l_ref, w2r_ref,
              y_ref, z_ref):
    inv = 1.0 / jnp.maximum(cnt_ref[0] + cnt_ref[1], 1.0)
    a1 = (p_ref[0] + p_ref[1]) * inv
    h2 = jnp.maximum(
        jnp.dot(a1, wl_ref[...], preferred_element_type=jnp.float32)
        + jnp.dot(h1_ref[...], wr_ref[...], preferred_element_type=jnp.float32)
        + b1_ref[...], 0.0)
    y_ref[...] = jnp.dot(h2, w2l_ref[...], preferred_element_type=jnp.float32)
    z_ref[...] = jnp.dot(h2, w2r_ref[...], preferred_element_type=jnp.float32)


def _tc_mid(h1, p, cnt, c1_Wl, c1_Wr, c1_b, c2_Wl, c2_Wr):
    return pl.pallas_call(
        _mid_body,
        grid=(N // BM,),
        in_specs=[
            pl.BlockSpec((BM, D), lambda i: (i, 0)),
            pl.BlockSpec((2, BM, D), lambda i: (0, i, 0)),
            pl.BlockSpec((2, BM, 1), lambda i: (0, i, 0)),
            pl.BlockSpec((D, 32), lambda i: (0, 0)),
            pl.BlockSpec((D, 32), lambda i: (0, 0)),
            pl.BlockSpec((1, 32), lambda i: (0, 0)),
            pl.BlockSpec((32, D), lambda i: (0, 0)),
            pl.BlockSpec((32, D), lambda i: (0, 0)),
        ],
        out_specs=[
            pl.BlockSpec((BM, D), lambda i: (i, 0)),
            pl.BlockSpec((BM, D), lambda i: (i, 0)),
        ],
        out_shape=[
            jax.ShapeDtypeStruct((N, D), jnp.float32),
            jax.ShapeDtypeStruct((N, D), jnp.float32),
        ],
    )(h1, p, cnt.reshape(2, N, 1), c1_Wl, c1_Wr, c1_b.reshape(1, 32), c2_Wl, c2_Wr)


def _fin_body(q_ref, cnt_ref, z_ref, b2_ref, w_ref, b_ref, o_ref):
    inv = 1.0 / jnp.maximum(cnt_ref[0] + cnt_ref[1], 1.0)
    a2 = (q_ref[0] + q_ref[1]) * inv
    h3 = jnp.maximum(a2 + z_ref[...] + b2_ref[...], 0.0)
    o = jnp.dot(h3, w_ref[...], preferred_element_type=jnp.float32) + b_ref[...]
    g = jax.nn.sigmoid(o[:, 1:2])
    fsi = jnp.maximum(o[:, 0:1], 0.0) + g
    mxi = jax.nn.sigmoid(o[:, 2:3])
    o_ref[...] = jnp.concatenate([fsi, g, mxi], axis=1)


def _tc_fin(q, cnt, z, c2_b, fc2_W, fc2_b):
    return pl.pallas_call(
        _fin_body,
        grid=(N // BM,),
        in_specs=[
            pl.BlockSpec((2, BM, D), lambda i: (0, i, 0)),
            pl.BlockSpec((2, BM, 1), lambda i: (0, i, 0)),
            pl.BlockSpec((BM, D), lambda i: (i, 0)),
            pl.BlockSpec((1, D), lambda i: (0, 0)),
            pl.BlockSpec((D, 3), lambda i: (0, 0)),
            pl.BlockSpec((1, 3), lambda i: (0, 0)),
        ],
        out_specs=pl.BlockSpec((BM, 3), lambda i: (i, 0)),
        out_shape=jax.ShapeDtypeStruct((N, 3), jnp.float32),
    )(q, cnt.reshape(2, N, 1), z, c2_b.reshape(1, D), fc2_W, fc2_b.reshape(1, 3))


def kernel(x, edge_index, fc1_W, fc1_b, c1_Wl, c1_Wr, c1_b,
           c2_Wl, c2_Wr, c2_b, fc2_W, fc2_b):
    pad = EPAD - E
    src2d = jnp.concatenate(
        [edge_index[0], jnp.zeros((pad,), jnp.int32)]).reshape(ROWS2D, CHUNK)
    dst2d = jnp.concatenate(
        [edge_index[1], jnp.full((pad,), N, jnp.int32)]).reshape(ROWS2D, CHUNK)
    h1 = _tc_fc1(x, fc1_W, fc1_b)
    p, cnt = _sc_agg_count()(h1, src2d, dst2d)
    y2, z2f = _tc_mid(h1, p, cnt, c1_Wl, c1_Wr, c1_b, c2_Wl, c2_Wr)
    q = _sc_agg()(y2, src2d, dst2d)
    return _tc_fin(q, cnt, z2f, c2_b, fc2_W, fc2_b)


# speedup vs baseline: 9.8960x; 1.0003x over previous
import functools

import jax
import jax.numpy as jnp
from jax import lax
from jax.experimental import pallas as pl
from jax.experimental.pallas import tpu as pltpu
from jax.experimental.pallas import tpu_sc as plsc

N = 100000
E = 1600000
D = 16
NP = 100008
CHUNK = 128
EPAD = 1638400
ROWS2D = EPAD // CHUNK
BLKE = 512
NB = 100
ZROWS = 6256
ZCAP = N - ZROWS
BM = 1000


def _agg_edges(h_hbm, src_hbm, dst_hbm, acc_sh, src_v, dst_v, rows_v,
               gsem, ssem, cnt_sh=None, ones_v=None):
    c = lax.axis_index("c")
    s = lax.axis_index("s")
    tile_e0 = (c * 16 + s) * (NB * BLKE)

    def gather_descs(bb):
        return [pltpu.make_async_copy(
            h_hbm.at[src_v.at[bb]], rows_v.at[bb], gsem.at[bb])]

    def scatter_descs(bb):
        out = [pltpu.make_async_copy(
            rows_v.at[bb], acc_sh.at[dst_v.at[bb]], ssem.at[bb])]
        if cnt_sh is not None:
            out.append(pltpu.make_async_copy(
                ones_v, cnt_sh.at[dst_v.at[bb]], ssem.at[bb]))
        return out

    def body(b, carry):
        bb = b % 2
        e0 = tile_e0 + b * BLKE

        @pl.when(b >= 2)
        def _():
            for d in scatter_descs(bb):
                d.wait()

        pltpu.sync_copy(src_hbm.at[pl.ds(e0, BLKE)], src_v.at[bb])
        pltpu.sync_copy(dst_hbm.at[pl.ds(e0, BLKE)], dst_v.at[bb])
        for d in gather_descs(bb):
            d.start()
        for d in gather_descs(bb):
            d.wait()
        for d in scatter_descs(bb):
            d.start(add=True)
        return carry

    lax.fori_loop(0, NB, body, 0)
    for bb in (0, 1):
        for d in scatter_descs(bb):
            d.wait()


ZB = 256


def _fill_zeros_2d(zv):
    def fz(i, c):
        zv[i, :] = jnp.zeros((16,), jnp.float32)
        return c
    lax.fori_loop(0, ZB, fz, 0)


def _fill_zeros_1d(zv):
    def fz(i, c):
        zv[pl.ds(i * 16, 16)] = jnp.zeros((16,), jnp.float32)
        return c
    lax.fori_loop(0, ZB // 16, fz, 0)


def _zero_slice_2d(acc_sh, zv, o):
    for i in range(ZROWS // ZB):
        pltpu.sync_copy(zv, acc_sh.at[pl.ds(o + i * ZB, ZB)])
    rem = ZROWS % ZB
    pltpu.sync_copy(zv.at[pl.ds(0, rem)], acc_sh.at[pl.ds(o + ZROWS - rem, rem)])


def _zero_slice_1d(cnt_sh, zv, o):
    for i in range(ZROWS // ZB):
        pltpu.sync_copy(zv, cnt_sh.at[pl.ds(o + i * ZB, ZB)])
    rem = ZROWS % ZB
    pltpu.sync_copy(zv.at[pl.ds(0, rem)], cnt_sh.at[pl.ds(o + ZROWS - rem, rem)])


@functools.lru_cache(maxsize=None)
def _sc_agg_count():
    mesh = plsc.VectorSubcoreMesh(core_axis_name="c", subcore_axis_name="s")

    @functools.partial(
        pl.kernel,
        mesh=mesh,
        compiler_params=pltpu.CompilerParams(use_tc_tiling_on_sc=False),
        out_type=(
            jax.ShapeDtypeStruct((2, N, D), jnp.float32),
            jax.ShapeDtypeStruct((2 * N,), jnp.float32),
        ),
        scratch_types=[
            pltpu.VMEM((2, BLKE), jnp.int32),
            pltpu.VMEM((2, BLKE), jnp.int32),
            pltpu.VMEM((2, BLKE, D), jnp.float32),
            pltpu.VMEM((BLKE,), jnp.float32),
            pltpu.VMEM((ZB, D), jnp.float32),
            pltpu.VMEM((ZB,), jnp.float32),
            pltpu.VMEM_SHARED((NP, D), jnp.float32),
            pltpu.VMEM_SHARED((NP,), jnp.float32),
            pltpu.SemaphoreType.DMA((2,)),
            pltpu.SemaphoreType.DMA((2,)),
        ],
    )
    def k(h_hbm, src_hbm, dst_hbm, p_hbm, cnt_hbm,
          src_v, dst_v, rows_v, ones_v, zv2d, zv1d, acc_sh, cnt_sh, gsem, ssem):
        c = lax.axis_index("c")
        s = lax.axis_index("s")
        o = jnp.minimum(s * ZROWS, ZCAP)
        _fill_zeros_2d(zv2d)
        _fill_zeros_1d(zv1d)
        _zero_slice_2d(acc_sh, zv2d, o)
        _zero_slice_1d(cnt_sh, zv1d, o)
        for i in range(BLKE // 16):
            ones_v[pl.ds(i * 16, 16)] = jnp.ones((16,), jnp.float32)
        plsc.subcore_barrier()
        _agg_edges(h_hbm, src_hbm, dst_hbm, acc_sh, src_v, dst_v, rows_v,
                   gsem, ssem, cnt_sh=cnt_sh, ones_v=ones_v)
        plsc.subcore_barrier()
        pltpu.sync_copy(acc_sh.at[pl.ds(o, ZROWS)], p_hbm.at[c, pl.ds(o, ZROWS)])
        pltpu.sync_copy(cnt_sh.at[pl.ds(o, ZROWS)], cnt_hbm.at[pl.ds(c * N + o, ZROWS)])

    return k


@functools.lru_cache(maxsize=None)
def _sc_agg():
    mesh = plsc.VectorSubcoreMesh(core_axis_name="c", subcore_axis_name="s")

    @functools.partial(
        pl.kernel,
        mesh=mesh,
        compiler_params=pltpu.CompilerParams(use_tc_tiling_on_sc=False),
        out_type=jax.ShapeDtypeStruct((2, N, D), jnp.float32),
        scratch_types=[
            pltpu.VMEM((2, BLKE), jnp.int32),
            pltpu.VMEM((2, BLKE), jnp.int32),
            pltpu.VMEM((2, BLKE, D), jnp.float32),
            pltpu.VMEM((ZB, D), jnp.float32),
            pltpu.VMEM_SHARED((NP, D), jnp.float32),
            pltpu.SemaphoreType.DMA((2,)),
            pltpu.SemaphoreType.DMA((2,)),
        ],
    )
    def k(h_hbm, src_hbm, dst_hbm, p_hbm,
          src_v, dst_v, rows_v, zv2d, acc_sh, gsem, ssem):
        c = lax.axis_index("c")
        s = lax.axis_index("s")
        o = jnp.minimum(s * ZROWS, ZCAP)
        _fill_zeros_2d(zv2d)
        _zero_slice_2d(acc_sh, zv2d, o)
        plsc.subcore_barrier()
        _agg_edges(h_hbm, src_hbm, dst_hbm, acc_sh, src_v, dst_v, rows_v,
                   gsem, ssem)
        plsc.subcore_barrier()
        pltpu.sync_copy(acc_sh.at[pl.ds(o, ZROWS)], p_hbm.at[c, pl.ds(o, ZROWS)])

    return k


def _fc1_body(x_ref, w_ref, b_ref, o_ref):
    o_ref[...] = jnp.maximum(
        jnp.dot(x_ref[...], w_ref[...], preferred_element_type=jnp.float32)
        + b_ref[...], 0.0)


def _tc_fc1(x, W, b):
    return pl.pallas_call(
        _fc1_body,
        grid=(N // BM,),
        in_specs=[
            pl.BlockSpec((BM, 128), lambda i: (i, 0)),
            pl.BlockSpec((128, D), lambda i: (0, 0)),
            pl.BlockSpec((1, D), lambda i: (0, 0)),
        ],
        out_specs=pl.BlockSpec((BM, D), lambda i: (i, 0)),
        out_shape=jax.ShapeDtypeStruct((N, D), jnp.float32),
    )(x, W, b.reshape(1, D))


def _mid_body(h1_ref, p_ref, cnt_ref, wl_ref, wr_ref, b1_ref, w2l_ref, w2r_ref,
              y_ref, z_ref):
    inv = 1.0 / jnp.maximum(cnt_ref[0] + cnt_ref[1], 1.0)
    a1 = (p_ref[0] + p_ref[1]) * inv
    h2 = jnp.maximum(
        jnp.dot(a1, wl_ref[...], preferred_element_type=jnp.float32)
        + jnp.dot(h1_ref[...], wr_ref[...], preferred_element_type=jnp.float32)
        + b1_ref[...], 0.0)
    y_ref[...] = jnp.dot(h2, w2l_ref[...], preferred_element_type=jnp.float32)
    z_ref[...] = jnp.dot(h2, w2r_ref[...], preferred_element_type=jnp.float32)


def _tc_mid(h1, p, cnt, c1_Wl, c1_Wr, c1_b, c2_Wl, c2_Wr):
    return pl.pallas_call(
        _mid_body,
        grid=(N // BM,),
        in_specs=[
            pl.BlockSpec((BM, D), lambda i: (i, 0)),
            pl.BlockSpec((2, BM, D), lambda i: (0, i, 0)),
            pl.BlockSpec((2, BM, 1), lambda i: (0, i, 0)),
            pl.BlockSpec((D, 32), lambda i: (0, 0)),
            pl.BlockSpec((D, 32), lambda i: (0, 0)),
            pl.BlockSpec((1, 32), lambda i: (0, 0)),
            pl.BlockSpec((32, D), lambda i: (0, 0)),
            pl.BlockSpec((32, D), lambda i: (0, 0)),
        ],
        out_specs=[
            pl.BlockSpec((BM, D), lambda i: (i, 0)),
            pl.BlockSpec((BM, D), lambda i: (i, 0)),
        ],
        out_shape=[
            jax.ShapeDtypeStruct((N, D), jnp.float32),
            jax.ShapeDtypeStruct((N, D), jnp.float32),
        ],
    )(h1, p, cnt.reshape(2, N, 1), c1_Wl, c1_Wr, c1_b.reshape(1, 32), c2_Wl, c2_Wr)


def _fin_body(q_ref, cnt_ref, z_ref, b2_ref, w_ref, b_ref, o_ref):
    inv = 1.0 / jnp.maximum(cnt_ref[0] + cnt_ref[1], 1.0)
    a2 = (q_ref[0] + q_ref[1]) * inv
    h3 = jnp.maximum(a2 + z_ref[...] + b2_ref[...], 0.0)
    o = jnp.dot(h3, w_ref[...], preferred_element_type=jnp.float32) + b_ref[...]
    g = jax.nn.sigmoid(o[:, 1:2])
    fsi = jnp.maximum(o[:, 0:1], 0.0) + g
    mxi = jax.nn.sigmoid(o[:, 2:3])
    o_ref[...] = jnp.concatenate([fsi, g, mxi], axis=1)


def _tc_fin(q, cnt, z, c2_b, fc2_W, fc2_b):
    return pl.pallas_call(
        _fin_body,
        grid=(N // BM,),
        in_specs=[
            pl.BlockSpec((2, BM, D), lambda i: (0, i, 0)),
            pl.BlockSpec((2, BM, 1), lambda i: (0, i, 0)),
            pl.BlockSpec((BM, D), lambda i: (i, 0)),
            pl.BlockSpec((1, D), lambda i: (0, 0)),
            pl.BlockSpec((D, 3), lambda i: (0, 0)),
            pl.BlockSpec((1, 3), lambda i: (0, 0)),
        ],
        out_specs=pl.BlockSpec((BM, 3), lambda i: (i, 0)),
        out_shape=jax.ShapeDtypeStruct((N, 3), jnp.float32),
    )(q, cnt.reshape(2, N, 1), z, c2_b.reshape(1, D), fc2_W, fc2_b.reshape(1, 3))


def kernel(x, edge_index, fc1_W, fc1_b, c1_Wl, c1_Wr, c1_b,
           c2_Wl, c2_Wr, c2_b, fc2_W, fc2_b):
    pad = EPAD - E
    src2d = jnp.concatenate([edge_index[0], jnp.zeros((pad,), jnp.int32)])
    dst2d = jnp.concatenate([edge_index[1], jnp.full((pad,), N, jnp.int32)])
    h1 = _tc_fc1(x, fc1_W, fc1_b)
    p, cnt = _sc_agg_count()(h1, src2d, dst2d)
    y2, z2f = _tc_mid(h1, p, cnt, c1_Wl, c1_Wr, c1_b, c2_Wl, c2_Wr)
    q = _sc_agg()(y2, src2d, dst2d)
    return _tc_fin(q, cnt, z2f, c2_b, fc2_W, fc2_b)


# speedup vs baseline: 12.4007x; 1.2531x over previous
import functools

import jax
import jax.numpy as jnp
from jax import lax
from jax.experimental import pallas as pl
from jax.experimental.pallas import tpu as pltpu
from jax.experimental.pallas import tpu_sc as plsc

N = 100000
E = 1600000
D = 16
PACK = 8
RN = N // PACK
NP = 100008
RNP = NP * D // 128
BLKE = 512
NB = 100
EPAD = 32 * NB * BLKE
PZ = 784
PCAP = RN - PZ
ZROWS = 6256
ZCAP = N - ZROWS
ZB = 256
BR = 512
GRID = (RN + BR - 1) // BR


def _fill_zeros_2d(zv):
    def fz(i, c):
        zv[i, :] = jnp.zeros((16,), jnp.float32)
        return c
    lax.fori_loop(0, ZB, fz, 0)


def _zero_slice_2d(acc_sh, zv, o):
    for i in range(ZROWS // ZB):
        pltpu.sync_copy(zv, acc_sh.at[pl.ds(o + i * ZB, ZB)])
    rem = ZROWS % ZB
    pltpu.sync_copy(zv.at[pl.ds(0, rem)], acc_sh.at[pl.ds(o + ZROWS - rem, rem)])


def _copy_out(acc_sh, out_hbm, c, s):
    o = jnp.minimum(s * ZROWS, ZCAP)
    pltpu.sync_copy(acc_sh.at[pl.ds(o, ZROWS)], out_hbm.at[c, pl.ds(o, ZROWS)])


@functools.lru_cache(maxsize=None)
def _sc_agg():
    mesh = plsc.VectorSubcoreMesh(core_axis_name="c", subcore_axis_name="s")

    @functools.partial(
        pl.kernel,
        mesh=mesh,
        compiler_params=pltpu.CompilerParams(use_tc_tiling_on_sc=False),
        out_type=jax.ShapeDtypeStruct((2, N, D), jnp.float32),
        scratch_types=[
            pltpu.VMEM((2, BLKE), jnp.int32),
            pltpu.VMEM((2, BLKE), jnp.int32),
            pltpu.VMEM((2, BLKE, D), jnp.float32),
            pltpu.VMEM((ZB, D), jnp.float32),
            pltpu.VMEM_SHARED((NP, D), jnp.float32),
            pltpu.SemaphoreType.DMA((2,)),
            pltpu.SemaphoreType.DMA((2,)),
        ],
    )
    def k(h_hbm, src_hbm, dst_hbm, p_hbm,
          src_v, dst_v, rows_v, zv2d, acc_sh, gsem, ssem):
        c = lax.axis_index("c")
        s = lax.axis_index("s")
        o = jnp.minimum(s * ZROWS, ZCAP)
        _fill_zeros_2d(zv2d)
        _zero_slice_2d(acc_sh, zv2d, o)
        plsc.subcore_barrier()

        tile_e0 = (c * 16 + s) * (NB * BLKE)

        def gather_desc(bb):
            return pltpu.make_async_copy(
                h_hbm.at[src_v.at[bb]], rows_v.at[bb], gsem.at[bb])

        def scatter_desc(bb):
            return pltpu.make_async_copy(
                rows_v.at[bb], acc_sh.at[dst_v.at[bb]], ssem.at[bb])

        def body(b, carry):
            bb = b % 2
            e0 = tile_e0 + b * BLKE

            @pl.when(b >= 2)
            def _():
                scatter_desc(bb).wait()

            pltpu.sync_copy(src_hbm.at[pl.ds(e0, BLKE)], src_v.at[bb])
            pltpu.sync_copy(dst_hbm.at[pl.ds(e0, BLKE)], dst_v.at[bb])
            gather_desc(bb).start()
            gather_desc(bb).wait()
            scatter_desc(bb).start(add=True)
            return carry

        lax.fori_loop(0, NB, body, 0)
        for bb in (0, 1):
            scatter_desc(bb).wait()
        plsc.subcore_barrier()
        _copy_out(acc_sh, p_hbm, c, s)

    return k


@functools.lru_cache(maxsize=None)
def _sc_count():
    mesh = plsc.VectorSubcoreMesh(core_axis_name="c", subcore_axis_name="s")

    @functools.partial(
        pl.kernel,
        mesh=mesh,
        compiler_params=pltpu.CompilerParams(use_tc_tiling_on_sc=False),
        out_type=jax.ShapeDtypeStruct((2, N, D), jnp.float32),
        scratch_types=[
            pltpu.VMEM((2, BLKE), jnp.int32),
            pltpu.VMEM((BLKE, D), jnp.float32),
            pltpu.VMEM((ZB, D), jnp.float32),
            pltpu.VMEM_SHARED((NP, D), jnp.float32),
            pltpu.SemaphoreType.DMA((2,)),
        ],
    )
    def k(dst_hbm, cnt_hbm, dst_v, ones_v, zv2d, acc_sh, ssem):
        c = lax.axis_index("c")
        s = lax.axis_index("s")
        o = jnp.minimum(s * ZROWS, ZCAP)
        _fill_zeros_2d(zv2d)
        _zero_slice_2d(acc_sh, zv2d, o)

        def fo(i, carry):
            ones_v[i, :] = jnp.ones((16,), jnp.float32)
            return carry
        lax.fori_loop(0, BLKE, fo, 0)
        plsc.subcore_barrier()

        tile_e0 = (c * 16 + s) * (NB * BLKE)

        def scatter_desc(bb):
            return pltpu.make_async_copy(
                ones_v, acc_sh.at[dst_v.at[bb]], ssem.at[bb])

        def body(b, carry):
            bb = b % 2

            @pl.when(b >= 2)
            def _():
                scatter_desc(bb).wait()

            pltpu.sync_copy(dst_hbm.at[pl.ds(tile_e0 + b * BLKE, BLKE)],
                            dst_v.at[bb])
            scatter_desc(bb).start(add=True)
            return carry

        lax.fori_loop(0, NB, body, 0)
        for bb in (0, 1):
            scatter_desc(bb).wait()
        plsc.subcore_barrier()
        _copy_out(acc_sh, cnt_hbm, c, s)

    return k


def _fc1_body(x_ref, w_ref, b_ref, o_ref):
    parts = [
        jnp.maximum(
            jnp.dot(x_ref[k], w_ref[...], preferred_element_type=jnp.float32)
            + b_ref[...], 0.0)
        for k in range(PACK)
    ]
    o_ref[...] = jnp.concatenate(parts, axis=1)


def _tc_fc1(x, W, b):
    return pl.pallas_call(
        _fc1_body,
        grid=(GRID,),
        in_specs=[
            pl.BlockSpec((PACK, BR, 128), lambda i: (0, i, 0)),
            pl.BlockSpec((128, D), lambda i: (0, 0)),
            pl.BlockSpec((1, D), lambda i: (0, 0)),
        ],
        out_specs=pl.BlockSpec((BR, 128), lambda i: (i, 0)),
        out_shape=jax.ShapeDtypeStruct((RN, 128), jnp.float32),
    )(x.reshape(PACK, RN, 128), W, b.reshape(1, D))


def _mid_body(h1_ref, p_ref, cnt_ref, wl_ref, wr_ref, b1_ref, w2l_ref, w2r_ref,
              y_ref, z_ref):
    inv = 1.0 / jnp.maximum(cnt_ref[0] + cnt_ref[1], 1.0)
    a1 = (p_ref[0] + p_ref[1]) * inv
    h2 = jnp.maximum(
        jnp.dot(a1, wl_ref[...], preferred_element_type=jnp.float32)
        + jnp.dot(h1_ref[...], wr_ref[...], preferred_element_type=jnp.float32)
        + b1_ref[...], 0.0)
    y_ref[...] = jnp.dot(h2, w2l_ref[...], preferred_element_type=jnp.float32)
    z_ref[...] = jnp.dot(h2, w2r_ref[...], preferred_element_type=jnp.float32)


def _tc_mid(h1p, p, cnt, Wl8, Wr8, b18, W2l8, W2r8):
    return pl.pallas_call(
        _mid_body,
        grid=(GRID,),
        in_specs=[
            pl.BlockSpec((BR, 128), lambda i: (i, 0)),
            pl.BlockSpec((2, BR, 128), lambda i: (0, i, 0)),
            pl.BlockSpec((2, BR, 128), lambda i: (0, i, 0)),
            pl.BlockSpec((128, 256), lambda i: (0, 0)),
            pl.BlockSpec((128, 256), lambda i: (0, 0)),
            pl.BlockSpec((1, 256), lambda i: (0, 0)),
            pl.BlockSpec((256, 128), lambda i: (0, 0)),
            pl.BlockSpec((256, 128), lambda i: (0, 0)),
        ],
        out_specs=[
            pl.BlockSpec((BR, 128), lambda i: (i, 0)),
            pl.BlockSpec((BR, 128), lambda i: (i, 0)),
        ],
        out_shape=[
            jax.ShapeDtypeStruct((RN, 128), jnp.float32),
            jax.ShapeDtypeStruct((RN, 128), jnp.float32),
        ],
    )(h1p, p, cnt, Wl8, Wr8, b18, W2l8, W2r8)


def _fin_body(q_ref, cnt_ref, z_ref, b28_ref, w8_ref, b8_ref, o_ref):
    inv = 1.0 / jnp.maximum(cnt_ref[0] + cnt_ref[1], 1.0)
    a2 = (q_ref[0] + q_ref[1]) * inv
    h3 = jnp.maximum(a2 + z_ref[...] + b28_ref[...], 0.0)
    o = jnp.dot(h3, w8_ref[...], preferred_element_type=jnp.float32) + b8_ref[...]
    parts = []
    for k in range(PACK):
        ok = o[:, 3 * k:3 * k + 3]
        g = jax.nn.sigmoid(ok[:, 1:2])
        fsi = jnp.maximum(ok[:, 0:1], 0.0) + g
        mxi = jax.nn.sigmoid(ok[:, 2:3])
        parts.append(jnp.concatenate([fsi, g, mxi], axis=1))
    o_ref[...] = jnp.concatenate(parts, axis=1)


def _tc_fin(q, cnt, z, b28, W8, b8):
    return pl.pallas_call(
        _fin_body,
        grid=(GRID,),
        in_specs=[
            pl.BlockSpec((2, BR, 128), lambda i: (0, i, 0)),
            pl.BlockSpec((2, BR, 128), lambda i: (0, i, 0)),
            pl.BlockSpec((BR, 128), lambda i: (i, 0)),
            pl.BlockSpec((1, 128), lambda i: (0, 0)),
            pl.BlockSpec((128, 24), lambda i: (0, 0)),
            pl.BlockSpec((1, 24), lambda i: (0, 0)),
        ],
        out_specs=pl.BlockSpec((BR, 24), lambda i: (i, 0)),
        out_shape=jax.ShapeDtypeStruct((RN, 24), jnp.float32),
    )(q, cnt, z, b28, W8, b8)


def kernel(x, edge_index, fc1_W, fc1_b, c1_Wl, c1_Wr, c1_b,
           c2_Wl, c2_Wr, c2_b, fc2_W, fc2_b):
    pad = EPAD - E
    src = edge_index[0]
    dst = edge_index[1]
    srcm = jnp.concatenate(
        [(src % RN) * PACK + src // RN, jnp.zeros((pad,), jnp.int32)])
    dstm = jnp.concatenate(
        [(dst % RN) * PACK + dst // RN, jnp.full((pad,), N, jnp.int32)])

    eye8 = jnp.eye(PACK, dtype=jnp.float32)
    Wl8 = jnp.kron(eye8, c1_Wl)
    Wr8 = jnp.kron(eye8, c1_Wr)
    W2l8 = jnp.kron(eye8, c2_Wl)
    W2r8 = jnp.kron(eye8, c2_Wr)
    W8 = jnp.kron(eye8, fc2_W)
    b18 = jnp.tile(c1_b, PACK).reshape(1, 256)
    b28 = jnp.tile(c2_b, PACK).reshape(1, 128)
    b8 = jnp.tile(fc2_b, PACK).reshape(1, 24)

    cnt = _sc_count()(dstm).reshape(2, RN, 128)
    h1p = _tc_fc1(x, fc1_W, fc1_b)
    p = _sc_agg()(h1p.reshape(N, D), srcm, dstm).reshape(2, RN, 128)
    y2p, z2p = _tc_mid(h1p, p, cnt, Wl8, Wr8, b18, W2l8, W2r8)
    q = _sc_agg()(y2p.reshape(N, D), srcm, dstm).reshape(2, RN, 128)
    out24 = _tc_fin(q, cnt, z2p, b28, W8, b8)
    return out24.reshape(RN, PACK, 3).transpose(1, 0, 2).reshape(N, 3)


# speedup vs baseline: 12.7048x; 1.0245x over previous
import functools

import jax
import jax.numpy as jnp
from jax import lax
from jax.experimental import pallas as pl
from jax.experimental.pallas import tpu as pltpu
from jax.experimental.pallas import tpu_sc as plsc

N = 100000
E = 1600000
D = 16
PACK = 8
RN = N // PACK
NP = 100008
RNP = NP * D // 128
BLKE = 512
NB = 100
EPAD = 32 * NB * BLKE
PZ = 784
PCAP = RN - PZ
ZROWS = 6256
ZCAP = N - ZROWS
ZB = 256
BR = 512
GRID = (RN + BR - 1) // BR


def _fill_zeros_2d(zv):
    def fz(i, c):
        zv[i, :] = jnp.zeros((16,), jnp.float32)
        return c
    lax.fori_loop(0, ZB, fz, 0)


def _zero_slice_2d(acc_sh, zv, o):
    for i in range(ZROWS // ZB):
        pltpu.sync_copy(zv, acc_sh.at[pl.ds(o + i * ZB, ZB)])
    rem = ZROWS % ZB
    pltpu.sync_copy(zv.at[pl.ds(0, rem)], acc_sh.at[pl.ds(o + ZROWS - rem, rem)])


def _copy_out(acc_sh, out_hbm, c, s):
    o = jnp.minimum(s * ZROWS, ZCAP)
    pltpu.sync_copy(acc_sh.at[pl.ds(o, ZROWS)], out_hbm.at[c, pl.ds(o, ZROWS)])


@functools.lru_cache(maxsize=None)
def _sc_agg():
    mesh = plsc.VectorSubcoreMesh(core_axis_name="c", subcore_axis_name="s")

    @functools.partial(
        pl.kernel,
        mesh=mesh,
        compiler_params=pltpu.CompilerParams(use_tc_tiling_on_sc=False),
        out_type=jax.ShapeDtypeStruct((2, N, D), jnp.float32),
        scratch_types=[
            pltpu.VMEM((2, BLKE), jnp.int32),
            pltpu.VMEM((2, BLKE), jnp.int32),
            pltpu.VMEM((2, BLKE, D), jnp.float32),
            pltpu.VMEM((ZB, D), jnp.float32),
            pltpu.VMEM_SHARED((NP, D), jnp.float32),
            pltpu.SemaphoreType.DMA((2,)),
            pltpu.SemaphoreType.DMA((2,)),
        ],
    )
    def k(h_hbm, src_hbm, dst_hbm, p_hbm,
          src_v, dst_v, rows_v, zv2d, acc_sh, gsem, ssem):
        c = lax.axis_index("c")
        s = lax.axis_index("s")
        o = jnp.minimum(s * ZROWS, ZCAP)
        _fill_zeros_2d(zv2d)
        _zero_slice_2d(acc_sh, zv2d, o)
        plsc.subcore_barrier()

        tile_e0 = (c * 16 + s) * (NB * BLKE)

        def gather_desc(bb):
            return pltpu.make_async_copy(
                h_hbm.at[src_v.at[bb]], rows_v.at[bb], gsem.at[bb])

        def scatter_desc(bb):
            return pltpu.make_async_copy(
                rows_v.at[bb], acc_sh.at[dst_v.at[bb]], ssem.at[bb])

        def body(b, carry):
            bb = b % 2
            e0 = tile_e0 + b * BLKE

            @pl.when(b >= 2)
            def _():
                scatter_desc(bb).wait()

            pltpu.sync_copy(src_hbm.at[pl.ds(e0, BLKE)], src_v.at[bb])
            pltpu.sync_copy(dst_hbm.at[pl.ds(e0, BLKE)], dst_v.at[bb])
            gather_desc(bb).start()
            gather_desc(bb).wait()
            scatter_desc(bb).start(add=True)
            return carry

        lax.fori_loop(0, NB, body, 0)
        for bb in (0, 1):
            scatter_desc(bb).wait()
        plsc.subcore_barrier()
        _copy_out(acc_sh, p_hbm, c, s)

    return k


@functools.lru_cache(maxsize=None)
def _sc_count():
    mesh = plsc.VectorSubcoreMesh(core_axis_name="c", subcore_axis_name="s")

    @functools.partial(
        pl.kernel,
        mesh=mesh,
        compiler_params=pltpu.CompilerParams(use_tc_tiling_on_sc=False),
        out_type=jax.ShapeDtypeStruct((2, N, D), jnp.float32),
        scratch_types=[
            pltpu.VMEM((2, BLKE), jnp.int32),
            pltpu.VMEM((BLKE, D), jnp.float32),
            pltpu.VMEM((ZB, D), jnp.float32),
            pltpu.VMEM_SHARED((NP, D), jnp.float32),
            pltpu.SemaphoreType.DMA((2,)),
        ],
    )
    def k(dst_hbm, cnt_hbm, dst_v, ones_v, zv2d, acc_sh, ssem):
        c = lax.axis_index("c")
        s = lax.axis_index("s")
        o = jnp.minimum(s * ZROWS, ZCAP)
        _fill_zeros_2d(zv2d)
        _zero_slice_2d(acc_sh, zv2d, o)

        def fo(i, carry):
            ones_v[i, :] = jnp.ones((16,), jnp.float32)
            return carry
        lax.fori_loop(0, BLKE, fo, 0)
        plsc.subcore_barrier()

        tile_e0 = (c * 16 + s) * (NB * BLKE)

        def scatter_desc(bb):
            return pltpu.make_async_copy(
                ones_v, acc_sh.at[dst_v.at[bb]], ssem.at[bb])

        def body(b, carry):
            bb = b % 2

            @pl.when(b >= 2)
            def _():
                scatter_desc(bb).wait()

            pltpu.sync_copy(dst_hbm.at[pl.ds(tile_e0 + b * BLKE, BLKE)],
                            dst_v.at[bb])
            scatter_desc(bb).start(add=True)
            return carry

        lax.fori_loop(0, NB, body, 0)
        for bb in (0, 1):
            scatter_desc(bb).wait()
        plsc.subcore_barrier()
        _copy_out(acc_sh, cnt_hbm, c, s)

    return k


def _fc1_body(x_ref, w_ref, b_ref, o_ref):
    o_ref[...] = jnp.maximum(
        jnp.dot(x_ref[...], w_ref[...], preferred_element_type=jnp.float32)
        + b_ref[...], 0.0)


def _tc_fc1(x8, W1B, b128):
    return pl.pallas_call(
        _fc1_body,
        grid=(GRID,),
        in_specs=[
            pl.BlockSpec((BR, 1024), lambda i: (i, 0)),
            pl.BlockSpec((1024, 128), lambda i: (0, 0)),
            pl.BlockSpec((1, 128), lambda i: (0, 0)),
        ],
        out_specs=pl.BlockSpec((BR, 128), lambda i: (i, 0)),
        out_shape=jax.ShapeDtypeStruct((RN, 128), jnp.float32),
    )(x8, W1B, b128)


def _mid_body(h1_ref, p_ref, cnt_ref, wl_ref, wr_ref, b1_ref, w2l_ref, w2r_ref,
              y_ref, z_ref):
    inv = 1.0 / jnp.maximum(cnt_ref[0] + cnt_ref[1], 1.0)
    a1 = (p_ref[0] + p_ref[1]) * inv
    h2 = jnp.maximum(
        jnp.dot(a1, wl_ref[...], preferred_element_type=jnp.float32)
        + jnp.dot(h1_ref[...], wr_ref[...], preferred_element_type=jnp.float32)
        + b1_ref[...], 0.0)
    y_ref[...] = jnp.dot(h2, w2l_ref[...], preferred_element_type=jnp.float32)
    z_ref[...] = jnp.dot(h2, w2r_ref[...], preferred_element_type=jnp.float32)


def _tc_mid(h1p, p, cnt, Wl8, Wr8, b18, W2l8, W2r8):
    return pl.pallas_call(
        _mid_body,
        grid=(GRID,),
        in_specs=[
            pl.BlockSpec((BR, 128), lambda i: (i, 0)),
            pl.BlockSpec((2, BR, 128), lambda i: (0, i, 0)),
            pl.BlockSpec((2, BR, 128), lambda i: (0, i, 0)),
            pl.BlockSpec((128, 256), lambda i: (0, 0)),
            pl.BlockSpec((128, 256), lambda i: (0, 0)),
            pl.BlockSpec((1, 256), lambda i: (0, 0)),
            pl.BlockSpec((256, 128), lambda i: (0, 0)),
            pl.BlockSpec((256, 128), lambda i: (0, 0)),
        ],
        out_specs=[
            pl.BlockSpec((BR, 128), lambda i: (i, 0)),
            pl.BlockSpec((BR, 128), lambda i: (i, 0)),
        ],
        out_shape=[
            jax.ShapeDtypeStruct((RN, 128), jnp.float32),
            jax.ShapeDtypeStruct((RN, 128), jnp.float32),
        ],
    )(h1p, p, cnt, Wl8, Wr8, b18, W2l8, W2r8)


def _fin_body(q_ref, cnt_ref, z_ref, b28_ref, w8_ref, b8_ref, o_ref):
    inv = 1.0 / jnp.maximum(cnt_ref[0] + cnt_ref[1], 1.0)
    a2 = (q_ref[0] + q_ref[1]) * inv
    h3 = jnp.maximum(a2 + z_ref[...] + b28_ref[...], 0.0)
    o = jnp.dot(h3, w8_ref[...], preferred_element_type=jnp.float32) + b8_ref[...]
    parts = []
    for k in range(PACK):
        ok = o[:, 3 * k:3 * k + 3]
        g = jax.nn.sigmoid(ok[:, 1:2])
        fsi = jnp.maximum(ok[:, 0:1], 0.0) + g
        mxi = jax.nn.sigmoid(ok[:, 2:3])
        parts.append(jnp.concatenate([fsi, g, mxi], axis=1))
    o_ref[...] = jnp.concatenate(parts, axis=1)


def _tc_fin(q, cnt, z, b28, W8, b8):
    return pl.pallas_call(
        _fin_body,
        grid=(GRID,),
        in_specs=[
            pl.BlockSpec((2, BR, 128), lambda i: (0, i, 0)),
            pl.BlockSpec((2, BR, 128), lambda i: (0, i, 0)),
            pl.BlockSpec((BR, 128), lambda i: (i, 0)),
            pl.BlockSpec((1, 128), lambda i: (0, 0)),
            pl.BlockSpec((128, 24), lambda i: (0, 0)),
            pl.BlockSpec((1, 24), lambda i: (0, 0)),
        ],
        out_specs=pl.BlockSpec((BR, 24), lambda i: (i, 0)),
        out_shape=jax.ShapeDtypeStruct((RN, 24), jnp.float32),
    )(q, cnt, z, b28, W8, b8)


def kernel(x, edge_index, fc1_W, fc1_b, c1_Wl, c1_Wr, c1_b,
           c2_Wl, c2_Wr, c2_b, fc2_W, fc2_b):
    pad = EPAD - E
    srcm = jnp.concatenate([edge_index[0], jnp.zeros((pad,), jnp.int32)])
    dstm = jnp.concatenate([edge_index[1], jnp.full((pad,), N, jnp.int32)])

    eye8 = jnp.eye(PACK, dtype=jnp.float32)
    W1B = jnp.kron(eye8, fc1_W)
    b128 = jnp.tile(fc1_b, PACK).reshape(1, 128)
    Wl8 = jnp.kron(eye8, c1_Wl)
    Wr8 = jnp.kron(eye8, c1_Wr)
    W2l8 = jnp.kron(eye8, c2_Wl)
    W2r8 = jnp.kron(eye8, c2_Wr)
    W8 = jnp.kron(eye8, fc2_W)
    b18 = jnp.tile(c1_b, PACK).reshape(1, 256)
    b28 = jnp.tile(c2_b, PACK).reshape(1, 128)
    b8 = jnp.tile(fc2_b, PACK).reshape(1, 24)

    cnt = _sc_count()(dstm).reshape(2, RN, 128)
    h1p = _tc_fc1(x.reshape(RN, PACK * 128), W1B, b128)
    p = _sc_agg()(h1p.reshape(N, D), srcm, dstm).reshape(2, RN, 128)
    y2p, z2p = _tc_mid(h1p, p, cnt, Wl8, Wr8, b18, W2l8, W2r8)
    q = _sc_agg()(y2p.reshape(N, D), srcm, dstm).reshape(2, RN, 128)
    out24 = _tc_fin(q, cnt, z2p, b28, W8, b8)
    return out24.reshape(N, 3)


# speedup vs baseline: 14.2254x; 1.1197x over previous
import functools

import jax
import jax.numpy as jnp
from jax import lax
from jax.experimental import pallas as pl
from jax.experimental.pallas import tpu as pltpu
from jax.experimental.pallas import tpu_sc as plsc

N = 100000
E = 1600000
D = 16
PACK = 8
RN = N // PACK
NP = 100008
RNP = NP * D // 128
BLKE = 512
NB = 100
NB0 = 132
NB1 = 2 * NB - NB0
EPAD = 32 * NB * BLKE
PZ = 784
PCAP = RN - PZ
ZROWS = 6256
ZCAP = N - ZROWS
ZB = 256
BR = 512
GRID = (RN + BR - 1) // BR


def _fill_zeros_2d(zv):
    def fz(i, c):
        zv[i, :] = jnp.zeros((16,), jnp.float32)
        return c
    lax.fori_loop(0, ZB, fz, 0)


def _zero_slice_2d(acc_sh, zv, o):
    for i in range(ZROWS // ZB):
        pltpu.sync_copy(zv, acc_sh.at[pl.ds(o + i * ZB, ZB)])
    rem = ZROWS % ZB
    pltpu.sync_copy(zv.at[pl.ds(0, rem)], acc_sh.at[pl.ds(o + ZROWS - rem, rem)])


def _copy_out(acc_sh, out_hbm, c, s):
    o = jnp.minimum(s * ZROWS, ZCAP)
    pltpu.sync_copy(acc_sh.at[pl.ds(o, ZROWS)], out_hbm.at[c, pl.ds(o, ZROWS)])


@functools.lru_cache(maxsize=None)
def _sc_agg():
    mesh = plsc.VectorSubcoreMesh(core_axis_name="c", subcore_axis_name="s")

    @functools.partial(
        pl.kernel,
        mesh=mesh,
        compiler_params=pltpu.CompilerParams(use_tc_tiling_on_sc=False),
        out_type=jax.ShapeDtypeStruct((2, N, D), jnp.float32),
        scratch_types=[
            pltpu.VMEM((2, BLKE), jnp.int32),
            pltpu.VMEM((2, BLKE), jnp.int32),
            pltpu.VMEM((2, BLKE, D), jnp.float32),
            pltpu.VMEM((ZB, D), jnp.float32),
            pltpu.VMEM_SHARED((NP, D), jnp.float32),
            pltpu.SemaphoreType.DMA((2,)),
            pltpu.SemaphoreType.DMA((2,)),
        ],
    )
    def k(h_hbm, src_hbm, dst_hbm, p_hbm,
          src_v, dst_v, rows_v, zv2d, acc_sh, gsem, ssem):
        c = lax.axis_index("c")
        s = lax.axis_index("s")
        o = jnp.minimum(s * ZROWS, ZCAP)
        _fill_zeros_2d(zv2d)
        _zero_slice_2d(acc_sh, zv2d, o)
        plsc.subcore_barrier()

        nbc = jnp.where(c == 0, NB0, NB1)
        tile_e0 = (c * 16 * NB0 + s * nbc) * BLKE

        def gather_desc(bb):
            return pltpu.make_async_copy(
                h_hbm.at[src_v.at[bb]], rows_v.at[bb], gsem.at[bb])

        def scatter_desc(bb):
            return pltpu.make_async_copy(
                rows_v.at[bb], acc_sh.at[dst_v.at[bb]], ssem.at[bb])

        def body(b, carry):
            bb = b % 2
            e0 = tile_e0 + b * BLKE

            @pl.when(b >= 2)
            def _():
                scatter_desc(bb).wait()

            pltpu.sync_copy(src_hbm.at[pl.ds(e0, BLKE)], src_v.at[bb])
            pltpu.sync_copy(dst_hbm.at[pl.ds(e0, BLKE)], dst_v.at[bb])
            gather_desc(bb).start()
            gather_desc(bb).wait()
            scatter_desc(bb).start(add=True)
            return carry

        lax.fori_loop(0, nbc, body, 0)
        for bb in (0, 1):
            scatter_desc(bb).wait()
        plsc.subcore_barrier()
        _copy_out(acc_sh, p_hbm, c, s)

    return k


@functools.lru_cache(maxsize=None)
def _sc_count():
    mesh = plsc.VectorSubcoreMesh(core_axis_name="c", subcore_axis_name="s")

    @functools.partial(
        pl.kernel,
        mesh=mesh,
        compiler_params=pltpu.CompilerParams(use_tc_tiling_on_sc=False),
        out_type=jax.ShapeDtypeStruct((2, N, D), jnp.float32),
        scratch_types=[
            pltpu.VMEM((2, BLKE), jnp.int32),
            pltpu.VMEM((BLKE, D), jnp.float32),
            pltpu.VMEM((ZB, D), jnp.float32),
            pltpu.VMEM_SHARED((NP, D), jnp.float32),
            pltpu.SemaphoreType.DMA((2,)),
        ],
    )
    def k(dst_hbm, cnt_hbm, dst_v, ones_v, zv2d, acc_sh, ssem):
        c = lax.axis_index("c")
        s = lax.axis_index("s")
        o = jnp.minimum(s * ZROWS, ZCAP)
        _fill_zeros_2d(zv2d)
        _zero_slice_2d(acc_sh, zv2d, o)

        def fo(i, carry):
            ones_v[i, :] = jnp.ones((16,), jnp.float32)
            return carry
        lax.fori_loop(0, BLKE, fo, 0)
        plsc.subcore_barrier()

        nbc = jnp.where(c == 0, NB0, NB1)
        tile_e0 = (c * 16 * NB0 + s * nbc) * BLKE

        def scatter_desc(bb):
            return pltpu.make_async_copy(
                ones_v, acc_sh.at[dst_v.at[bb]], ssem.at[bb])

        def body(b, carry):
            bb = b % 2

            @pl.when(b >= 2)
            def _():
                scatter_desc(bb).wait()

            pltpu.sync_copy(dst_hbm.at[pl.ds(tile_e0 + b * BLKE, BLKE)],
                            dst_v.at[bb])
            scatter_desc(bb).start(add=True)
            return carry

        lax.fori_loop(0, nbc, body, 0)
        for bb in (0, 1):
            scatter_desc(bb).wait()
        plsc.subcore_barrier()
        _copy_out(acc_sh, cnt_hbm, c, s)

    return k


def _fc1_body(x_ref, w_ref, b_ref, o_ref):
    o_ref[...] = jnp.maximum(
        jnp.dot(x_ref[...], w_ref[...], preferred_element_type=jnp.float32)
        + b_ref[...], 0.0)


def _tc_fc1(x8, W1B, b128):
    return pl.pallas_call(
        _fc1_body,
        grid=(GRID,),
        in_specs=[
            pl.BlockSpec((BR, 1024), lambda i: (i, 0)),
            pl.BlockSpec((1024, 128), lambda i: (0, 0)),
            pl.BlockSpec((1, 128), lambda i: (0, 0)),
        ],
        out_specs=pl.BlockSpec((BR, 128), lambda i: (i, 0)),
        out_shape=jax.ShapeDtypeStruct((RN, 128), jnp.float32),
    )(x8, W1B, b128)


def _mid_body(h1_ref, p_ref, cnt_ref, wl_ref, wr_ref, b1_ref, w2l_ref, w2r_ref,
              y_ref, z_ref):
    inv = 1.0 / jnp.maximum(cnt_ref[0] + cnt_ref[1], 1.0)
    a1 = (p_ref[0] + p_ref[1]) * inv
    h2 = jnp.maximum(
        jnp.dot(a1, wl_ref[...], preferred_element_type=jnp.float32)
        + jnp.dot(h1_ref[...], wr_ref[...], preferred_element_type=jnp.float32)
        + b1_ref[...], 0.0)
    y_ref[...] = jnp.dot(h2, w2l_ref[...], preferred_element_type=jnp.float32)
    z_ref[...] = jnp.dot(h2, w2r_ref[...], preferred_element_type=jnp.float32)


def _tc_mid(h1p, p, cnt, Wl8, Wr8, b18, W2l8, W2r8):
    return pl.pallas_call(
        _mid_body,
        grid=(GRID,),
        in_specs=[
            pl.BlockSpec((BR, 128), lambda i: (i, 0)),
            pl.BlockSpec((2, BR, 128), lambda i: (0, i, 0)),
            pl.BlockSpec((2, BR, 128), lambda i: (0, i, 0)),
            pl.BlockSpec((128, 256), lambda i: (0, 0)),
            pl.BlockSpec((128, 256), lambda i: (0, 0)),
            pl.BlockSpec((1, 256), lambda i: (0, 0)),
            pl.BlockSpec((256, 128), lambda i: (0, 0)),
            pl.BlockSpec((256, 128), lambda i: (0, 0)),
        ],
        out_specs=[
            pl.BlockSpec((BR, 128), lambda i: (i, 0)),
            pl.BlockSpec((BR, 128), lambda i: (i, 0)),
        ],
        out_shape=[
            jax.ShapeDtypeStruct((RN, 128), jnp.float32),
            jax.ShapeDtypeStruct((RN, 128), jnp.float32),
        ],
    )(h1p, p, cnt, Wl8, Wr8, b18, W2l8, W2r8)


def _fin_body(q_ref, cnt_ref, z_ref, b28_ref, w8_ref, b8_ref, o_ref):
    inv = 1.0 / jnp.maximum(cnt_ref[0] + cnt_ref[1], 1.0)
    a2 = (q_ref[0] + q_ref[1]) * inv
    h3 = jnp.maximum(a2 + z_ref[...] + b28_ref[...], 0.0)
    o = jnp.dot(h3, w8_ref[...], preferred_element_type=jnp.float32) + b8_ref[...]
    parts = []
    for k in range(PACK):
        ok = o[:, 3 * k:3 * k + 3]
        g = jax.nn.sigmoid(ok[:, 1:2])
        fsi = jnp.maximum(ok[:, 0:1], 0.0) + g
        mxi = jax.nn.sigmoid(ok[:, 2:3])
        parts.append(jnp.concatenate([fsi, g, mxi], axis=1))
    o_ref[...] = jnp.concatenate(parts, axis=1)


def _tc_fin(q, cnt, z, b28, W8, b8):
    return pl.pallas_call(
        _fin_body,
        grid=(GRID,),
        in_specs=[
            pl.BlockSpec((2, BR, 128), lambda i: (0, i, 0)),
            pl.BlockSpec((2, BR, 128), lambda i: (0, i, 0)),
            pl.BlockSpec((BR, 128), lambda i: (i, 0)),
            pl.BlockSpec((1, 128), lambda i: (0, 0)),
            pl.BlockSpec((128, 24), lambda i: (0, 0)),
            pl.BlockSpec((1, 24), lambda i: (0, 0)),
        ],
        out_specs=pl.BlockSpec((BR, 24), lambda i: (i, 0)),
        out_shape=jax.ShapeDtypeStruct((RN, 24), jnp.float32),
    )(q, cnt, z, b28, W8, b8)


def kernel(x, edge_index, fc1_W, fc1_b, c1_Wl, c1_Wr, c1_b,
           c2_Wl, c2_Wr, c2_b, fc2_W, fc2_b):
    pad = EPAD - E
    srcm = jnp.concatenate([edge_index[0], jnp.zeros((pad,), jnp.int32)])
    dstm = jnp.concatenate([edge_index[1], jnp.full((pad,), N, jnp.int32)])

    eye8 = jnp.eye(PACK, dtype=jnp.float32)
    W1B = jnp.kron(eye8, fc1_W)
    b128 = jnp.tile(fc1_b, PACK).reshape(1, 128)
    Wl8 = jnp.kron(eye8, c1_Wl)
    Wr8 = jnp.kron(eye8, c1_Wr)
    W2l8 = jnp.kron(eye8, c2_Wl)
    W2r8 = jnp.kron(eye8, c2_Wr)
    W8 = jnp.kron(eye8, fc2_W)
    b18 = jnp.tile(c1_b, PACK).reshape(1, 256)
    b28 = jnp.tile(c2_b, PACK).reshape(1, 128)
    b8 = jnp.tile(fc2_b, PACK).reshape(1, 24)

    cnt = _sc_count()(dstm).reshape(2, RN, 128)
    h1p = _tc_fc1(x.reshape(RN, PACK * 128), W1B, b128)
    p = _sc_agg()(h1p.reshape(N, D), srcm, dstm).reshape(2, RN, 128)
    y2p, z2p = _tc_mid(h1p, p, cnt, Wl8, Wr8, b18, W2l8, W2r8)
    q = _sc_agg()(y2p.reshape(N, D), srcm, dstm).reshape(2, RN, 128)
    out24 = _tc_fin(q, cnt, z2p, b28, W8, b8)
    return out24.reshape(N, 3)


# speedup vs baseline: 17.2905x; 1.2155x over previous
import functools

import jax
import jax.numpy as jnp
from jax import lax
from jax.experimental import pallas as pl
from jax.experimental.pallas import tpu as pltpu
from jax.experimental.pallas import tpu_sc as plsc

N = 100000
E = 1600000
D = 16
PACK = 8
RN = N // PACK
NP = 100008
RNP = NP * D // 128
BLKE = 512
EBLK = E // BLKE
B0 = 2062
B1 = EBLK - B0
Q0, R0 = divmod(B0, 16)
Q1, R1 = divmod(B1, 16)
PZ = 784
PCAP = RN - PZ
ZROWS = 6256
ZCAP = N - ZROWS
ZB = 256
BR = 2504
GRID = (RN + BR - 1) // BR


def _fill_zeros_2d(zv):
    def fz(i, c):
        zv[i, :] = jnp.zeros((16,), jnp.float32)
        return c
    lax.fori_loop(0, ZB, fz, 0)


def _zero_slice_2d(acc_sh, zv, o):
    for i in range(ZROWS // ZB):
        pltpu.sync_copy(zv, acc_sh.at[pl.ds(o + i * ZB, ZB)])
    rem = ZROWS % ZB
    pltpu.sync_copy(zv.at[pl.ds(0, rem)], acc_sh.at[pl.ds(o + ZROWS - rem, rem)])


def _tile_blocks(c, s):
    q = jnp.where(c == 0, Q0, Q1)
    r = jnp.where(c == 0, R0, R1)
    nb = q + (s < r).astype(jnp.int32)
    bstart = jnp.where(c == 0, 0, B0) + s * q + jnp.minimum(s, r)
    return bstart, nb


def _copy_out(acc_sh, out_hbm, c, s):
    o = jnp.minimum(s * ZROWS, ZCAP)
    pltpu.sync_copy(acc_sh.at[pl.ds(o, ZROWS)], out_hbm.at[c, pl.ds(o, ZROWS)])


@functools.lru_cache(maxsize=None)
def _sc_agg():
    mesh = plsc.VectorSubcoreMesh(core_axis_name="c", subcore_axis_name="s")

    @functools.partial(
        pl.kernel,
        mesh=mesh,
        compiler_params=pltpu.CompilerParams(use_tc_tiling_on_sc=False),
        out_type=jax.ShapeDtypeStruct((2, N, D), jnp.float32),
        scratch_types=[
            pltpu.VMEM((2, BLKE), jnp.int32),
            pltpu.VMEM((2, BLKE), jnp.int32),
            pltpu.VMEM((2, BLKE, D), jnp.float32),
            pltpu.VMEM((ZB, D), jnp.float32),
            pltpu.VMEM_SHARED((NP, D), jnp.float32),
            pltpu.SemaphoreType.DMA((2,)),
            pltpu.SemaphoreType.DMA((2,)),
        ],
    )
    def k(h_hbm, ei_hbm, p_hbm,
          src_v, dst_v, rows_v, zv2d, acc_sh, gsem, ssem):
        c = lax.axis_index("c")
        s = lax.axis_index("s")
        o = jnp.minimum(s * ZROWS, ZCAP)
        _fill_zeros_2d(zv2d)
        _zero_slice_2d(acc_sh, zv2d, o)
        plsc.subcore_barrier()

        bstart, nbc = _tile_blocks(c, s)

        def gather_desc(bb):
            return pltpu.make_async_copy(
                h_hbm.at[src_v.at[bb]], rows_v.at[bb], gsem.at[bb])

        def scatter_desc(bb):
            return pltpu.make_async_copy(
                rows_v.at[bb], acc_sh.at[dst_v.at[bb]], ssem.at[bb])

        def body(b, carry):
            bb = b % 2
            e0 = (bstart + b) * BLKE

            @pl.when(b >= 2)
            def _():
                scatter_desc(bb).wait()

            pltpu.sync_copy(ei_hbm.at[0, pl.ds(e0, BLKE)], src_v.at[bb])
            pltpu.sync_copy(ei_hbm.at[1, pl.ds(e0, BLKE)], dst_v.at[bb])
            gather_desc(bb).start()
            gather_desc(bb).wait()
            scatter_desc(bb).start(add=True)
            return carry

        lax.fori_loop(0, nbc, body, 0)
        for bb in (0, 1):
            scatter_desc(bb).wait()
        plsc.subcore_barrier()
        _copy_out(acc_sh, p_hbm, c, s)

    return k


@functools.lru_cache(maxsize=None)
def _sc_count():
    mesh = plsc.VectorSubcoreMesh(core_axis_name="c", subcore_axis_name="s")

    @functools.partial(
        pl.kernel,
        mesh=mesh,
        compiler_params=pltpu.CompilerParams(use_tc_tiling_on_sc=False),
        out_type=jax.ShapeDtypeStruct((2, N, D), jnp.float32),
        scratch_types=[
            pltpu.VMEM((2, BLKE), jnp.int32),
            pltpu.VMEM((BLKE, D), jnp.float32),
            pltpu.VMEM((ZB, D), jnp.float32),
            pltpu.VMEM_SHARED((NP, D), jnp.float32),
            pltpu.SemaphoreType.DMA((2,)),
        ],
    )
    def k(ei_hbm, cnt_hbm, dst_v, ones_v, zv2d, acc_sh, ssem):
        c = lax.axis_index("c")
        s = lax.axis_index("s")
        o = jnp.minimum(s * ZROWS, ZCAP)
        _fill_zeros_2d(zv2d)
        _zero_slice_2d(acc_sh, zv2d, o)

        def fo(i, carry):
            ones_v[i, :] = jnp.ones((16,), jnp.float32)
            return carry
        lax.fori_loop(0, BLKE, fo, 0)
        plsc.subcore_barrier()

        bstart, nbc = _tile_blocks(c, s)

        def scatter_desc(bb):
            return pltpu.make_async_copy(
                ones_v, acc_sh.at[dst_v.at[bb]], ssem.at[bb])

        def body(b, carry):
            bb = b % 2

            @pl.when(b >= 2)
            def _():
                scatter_desc(bb).wait()

            pltpu.sync_copy(ei_hbm.at[1, pl.ds((bstart + b) * BLKE, BLKE)],
                            dst_v.at[bb])
            scatter_desc(bb).start(add=True)
            return carry

        lax.fori_loop(0, nbc, body, 0)
        for bb in (0, 1):
            scatter_desc(bb).wait()
        plsc.subcore_barrier()
        _copy_out(acc_sh, cnt_hbm, c, s)

    return k


def _fc1_body(x_ref, w_ref, b_ref, o_ref):
    o_ref[...] = jnp.maximum(
        jnp.dot(x_ref[...], w_ref[...], preferred_element_type=jnp.float32)
        + b_ref[...], 0.0)


def _tc_fc1(x8, W1B, b128):
    return pl.pallas_call(
        _fc1_body,
        grid=(GRID,),
        in_specs=[
            pl.BlockSpec((BR, 1024), lambda i: (i, 0)),
            pl.BlockSpec((1024, 128), lambda i: (0, 0)),
            pl.BlockSpec((1, 128), lambda i: (0, 0)),
        ],
        out_specs=pl.BlockSpec((BR, 128), lambda i: (i, 0)),
        out_shape=jax.ShapeDtypeStruct((RN, 128), jnp.float32),
    )(x8, W1B, b128)


def _mid_body(h1_ref, p_ref, cnt_ref, wl_ref, wr_ref, b1_ref, w2l_ref, w2r_ref,
              y_ref, z_ref):
    inv = 1.0 / jnp.maximum(cnt_ref[0] + cnt_ref[1], 1.0)
    a1 = (p_ref[0] + p_ref[1]) * inv
    h2 = jnp.maximum(
        jnp.dot(a1, wl_ref[...], preferred_element_type=jnp.float32)
        + jnp.dot(h1_ref[...], wr_ref[...], preferred_element_type=jnp.float32)
        + b1_ref[...], 0.0)
    y_ref[...] = jnp.dot(h2, w2l_ref[...], preferred_element_type=jnp.float32)
    z_ref[...] = jnp.dot(h2, w2r_ref[...], preferred_element_type=jnp.float32)


def _tc_mid(h1p, p, cnt, Wl8, Wr8, b18, W2l8, W2r8):
    return pl.pallas_call(
        _mid_body,
        grid=(GRID,),
        in_specs=[
            pl.BlockSpec((BR, 128), lambda i: (i, 0)),
            pl.BlockSpec((2, BR, 128), lambda i: (0, i, 0)),
            pl.BlockSpec((2, BR, 128), lambda i: (0, i, 0)),
            pl.BlockSpec((128, 256), lambda i: (0, 0)),
            pl.BlockSpec((128, 256), lambda i: (0, 0)),
            pl.BlockSpec((1, 256), lambda i: (0, 0)),
            pl.BlockSpec((256, 128), lambda i: (0, 0)),
            pl.BlockSpec((256, 128), lambda i: (0, 0)),
        ],
        out_specs=[
            pl.BlockSpec((BR, 128), lambda i: (i, 0)),
            pl.BlockSpec((BR, 128), lambda i: (i, 0)),
        ],
        out_shape=[
            jax.ShapeDtypeStruct((RN, 128), jnp.float32),
            jax.ShapeDtypeStruct((RN, 128), jnp.float32),
        ],
    )(h1p, p, cnt, Wl8, Wr8, b18, W2l8, W2r8)


def _fin_body(q_ref, cnt_ref, z_ref, b28_ref, w8_ref, b8_ref, o_ref):
    inv = 1.0 / jnp.maximum(cnt_ref[0] + cnt_ref[1], 1.0)
    a2 = (q_ref[0] + q_ref[1]) * inv
    h3 = jnp.maximum(a2 + z_ref[...] + b28_ref[...], 0.0)
    o = jnp.dot(h3, w8_ref[...], preferred_element_type=jnp.float32) + b8_ref[...]
    parts = []
    for k in range(PACK):
        ok = o[:, 3 * k:3 * k + 3]
        g = jax.nn.sigmoid(ok[:, 1:2])
        fsi = jnp.maximum(ok[:, 0:1], 0.0) + g
        mxi = jax.nn.sigmoid(ok[:, 2:3])
        parts.append(jnp.concatenate([fsi, g, mxi], axis=1))
    o_ref[...] = jnp.concatenate(parts, axis=1)


def _tc_fin(q, cnt, z, b28, W8, b8):
    return pl.pallas_call(
        _fin_body,
        grid=(GRID,),
        in_specs=[
            pl.BlockSpec((2, BR, 128), lambda i: (0, i, 0)),
            pl.BlockSpec((2, BR, 128), lambda i: (0, i, 0)),
            pl.BlockSpec((BR, 128), lambda i: (i, 0)),
            pl.BlockSpec((1, 128), lambda i: (0, 0)),
            pl.BlockSpec((128, 24), lambda i: (0, 0)),
            pl.BlockSpec((1, 24), lambda i: (0, 0)),
        ],
        out_specs=pl.BlockSpec((BR, 24), lambda i: (i, 0)),
        out_shape=jax.ShapeDtypeStruct((RN, 24), jnp.float32),
    )(q, cnt, z, b28, W8, b8)


def kernel(x, edge_index, fc1_W, fc1_b, c1_Wl, c1_Wr, c1_b,
           c2_Wl, c2_Wr, c2_b, fc2_W, fc2_b):
    eye8 = jnp.eye(PACK, dtype=jnp.float32)
    W1B = jnp.kron(eye8, fc1_W)
    b128 = jnp.tile(fc1_b, PACK).reshape(1, 128)
    Wl8 = jnp.kron(eye8, c1_Wl)
    Wr8 = jnp.kron(eye8, c1_Wr)
    W2l8 = jnp.kron(eye8, c2_Wl)
    W2r8 = jnp.kron(eye8, c2_Wr)
    W8 = jnp.kron(eye8, fc2_W)
    b18 = jnp.tile(c1_b, PACK).reshape(1, 256)
    b28 = jnp.tile(c2_b, PACK).reshape(1, 128)
    b8 = jnp.tile(fc2_b, PACK).reshape(1, 24)

    cnt = _sc_count()(edge_index).reshape(2, RN, 128)
    h1p = _tc_fc1(x.reshape(RN, PACK * 128), W1B, b128)
    p = _sc_agg()(h1p.reshape(N, D), edge_index).reshape(2, RN, 128)
    y2p, z2p = _tc_mid(h1p, p, cnt, Wl8, Wr8, b18, W2l8, W2r8)
    q = _sc_agg()(y2p.reshape(N, D), edge_index).reshape(2, RN, 128)
    out24 = _tc_fin(q, cnt, z2p, b28, W8, b8)
    return out24.reshape(N, 3)


# speedup vs baseline: 19.8978x; 1.1508x over previous
import functools

import jax
import jax.numpy as jnp
from jax import lax
from jax.experimental import pallas as pl
from jax.experimental.pallas import tpu as pltpu
from jax.experimental.pallas import tpu_sc as plsc

N = 100000
E = 1600000
D = 16
PACK = 8
RN = N // PACK
NP = 100008
RNP = NP * D // 128
BLKE = 512
EBLK = E // BLKE
B0 = 1575
B1 = EBLK - B0
Q0, R0 = divmod(B0, 16)
Q1, R1 = divmod(B1, 16)
PZ = 784
PCAP = RN - PZ
ZROWS = 6256
ZCAP = N - ZROWS
ZB = 256
BR = 2504
GRID = (RN + BR - 1) // BR


def _fill_zeros_2d(zv):
    def fz(i, c):
        zv[i, :] = jnp.zeros((16,), jnp.float32)
        return c
    lax.fori_loop(0, ZB, fz, 0)


def _zero_slice_2d(acc_sh, zv, o):
    for i in range(ZROWS // ZB):
        pltpu.sync_copy(zv, acc_sh.at[pl.ds(o + i * ZB, ZB)])
    rem = ZROWS % ZB
    pltpu.sync_copy(zv.at[pl.ds(0, rem)], acc_sh.at[pl.ds(o + ZROWS - rem, rem)])


def _tile_blocks(c, s):
    q = jnp.where(c == 0, Q0, Q1)
    r = jnp.where(c == 0, R0, R1)
    nb = q + (s < r).astype(jnp.int32)
    bstart = jnp.where(c == 0, 0, B0) + s * q + jnp.minimum(s, r)
    return bstart, nb


def _copy_out(acc_sh, out_hbm, c, s):
    o = jnp.minimum(s * ZROWS, ZCAP)
    pltpu.sync_copy(acc_sh.at[pl.ds(o, ZROWS)], out_hbm.at[c, pl.ds(o, ZROWS)])


@functools.lru_cache(maxsize=None)
def _sc_agg():
    mesh = plsc.VectorSubcoreMesh(core_axis_name="c", subcore_axis_name="s")

    @functools.partial(
        pl.kernel,
        mesh=mesh,
        compiler_params=pltpu.CompilerParams(use_tc_tiling_on_sc=False),
        out_type=jax.ShapeDtypeStruct((2, N, D), jnp.float32),
        scratch_types=[
            pltpu.VMEM((2, BLKE), jnp.int32),
            pltpu.VMEM((2, BLKE), jnp.int32),
            pltpu.VMEM((2, BLKE, D), jnp.float32),
            pltpu.VMEM((ZB, D), jnp.float32),
            pltpu.VMEM_SHARED((NP, D), jnp.float32),
            pltpu.SemaphoreType.DMA((2,)),
            pltpu.SemaphoreType.DMA((2,)),
        ],
    )
    def k(h_hbm, srce_hbm, dste_hbm, p_hbm,
          src_v, dst_v, rows_v, zv2d, acc_sh, gsem, ssem):
        c = lax.axis_index("c")
        s = lax.axis_index("s")
        o = jnp.minimum(s * ZROWS, ZCAP)
        _fill_zeros_2d(zv2d)
        _zero_slice_2d(acc_sh, zv2d, o)
        plsc.subcore_barrier()

        bstart, nbc = _tile_blocks(c, s)

        def gather_desc(bb):
            return pltpu.make_async_copy(
                h_hbm.at[src_v.at[bb]], rows_v.at[bb], gsem.at[bb])

        def scatter_desc(bb):
            return pltpu.make_async_copy(
                rows_v.at[bb], acc_sh.at[dst_v.at[bb]], ssem.at[bb])

        def body(b, carry):
            bb = b % 2
            e0 = (bstart + b) * BLKE

            @pl.when(b >= 2)
            def _():
                scatter_desc(bb).wait()

            pltpu.sync_copy(srce_hbm.at[pl.ds(e0, BLKE)], src_v.at[bb])
            pltpu.sync_copy(dste_hbm.at[pl.ds(e0, BLKE)], dst_v.at[bb])
            gather_desc(bb).start()
            gather_desc(bb).wait()
            scatter_desc(bb).start(add=True)
            return carry

        lax.fori_loop(0, nbc, body, 0)
        for bb in (0, 1):
            scatter_desc(bb).wait()
        plsc.subcore_barrier()
        _copy_out(acc_sh, p_hbm, c, s)

    return k


@functools.lru_cache(maxsize=None)
def _sc_count():
    mesh = plsc.VectorSubcoreMesh(core_axis_name="c", subcore_axis_name="s")

    @functools.partial(
        pl.kernel,
        mesh=mesh,
        compiler_params=pltpu.CompilerParams(use_tc_tiling_on_sc=False),
        out_type=jax.ShapeDtypeStruct((2, N, D), jnp.float32),
        scratch_types=[
            pltpu.VMEM((2, BLKE), jnp.int32),
            pltpu.VMEM((BLKE, D), jnp.float32),
            pltpu.VMEM((ZB, D), jnp.float32),
            pltpu.VMEM_SHARED((NP, D), jnp.float32),
            pltpu.SemaphoreType.DMA((2,)),
        ],
    )
    def k(dste_hbm, cnt_hbm, dst_v, ones_v, zv2d, acc_sh, ssem):
        c = lax.axis_index("c")
        s = lax.axis_index("s")
        o = jnp.minimum(s * ZROWS, ZCAP)
        _fill_zeros_2d(zv2d)
        _zero_slice_2d(acc_sh, zv2d, o)

        def fo(i, carry):
            ones_v[i, :] = jnp.ones((16,), jnp.float32)
            return carry
        lax.fori_loop(0, BLKE, fo, 0)
        plsc.subcore_barrier()

        bstart, nbc = _tile_blocks(c, s)

        def scatter_desc(bb):
            return pltpu.make_async_copy(
                ones_v, acc_sh.at[dst_v.at[bb]], ssem.at[bb])

        def body(b, carry):
            bb = b % 2

            @pl.when(b >= 2)
            def _():
                scatter_desc(bb).wait()

            pltpu.sync_copy(dste_hbm.at[pl.ds((bstart + b) * BLKE, BLKE)],
                            dst_v.at[bb])
            scatter_desc(bb).start(add=True)
            return carry

        lax.fori_loop(0, nbc, body, 0)
        for bb in (0, 1):
            scatter_desc(bb).wait()
        plsc.subcore_barrier()
        _copy_out(acc_sh, cnt_hbm, c, s)

    return k


def _fc1_body(x_ref, w_ref, b_ref, o_ref):
    o_ref[...] = jnp.maximum(
        jnp.dot(x_ref[...], w_ref[...], preferred_element_type=jnp.float32)
        + b_ref[...], 0.0)


def _tc_fc1(x8, W1B, b128):
    return pl.pallas_call(
        _fc1_body,
        grid=(GRID,),
        in_specs=[
            pl.BlockSpec((BR, 1024), lambda i: (i, 0)),
            pl.BlockSpec((1024, 128), lambda i: (0, 0)),
            pl.BlockSpec((1, 128), lambda i: (0, 0)),
        ],
        out_specs=pl.BlockSpec((BR, 128), lambda i: (i, 0)),
        out_shape=jax.ShapeDtypeStruct((RN, 128), jnp.float32),
    )(x8, W1B, b128)


def _mid_body(h1_ref, p_ref, cnt_ref, wl_ref, wr_ref, b1_ref, w2l_ref, w2r_ref,
              y_ref, z_ref):
    inv = 1.0 / jnp.maximum(cnt_ref[0] + cnt_ref[1], 1.0)
    a1 = (p_ref[0] + p_ref[1]) * inv
    h2 = jnp.maximum(
        jnp.dot(a1, wl_ref[...], preferred_element_type=jnp.float32)
        + jnp.dot(h1_ref[...], wr_ref[...], preferred_element_type=jnp.float32)
        + b1_ref[...], 0.0)
    y_ref[...] = jnp.dot(h2, w2l_ref[...], preferred_element_type=jnp.float32)
    z_ref[...] = jnp.dot(h2, w2r_ref[...], preferred_element_type=jnp.float32)


def _tc_mid(h1p, p, cnt, Wl8, Wr8, b18, W2l8, W2r8):
    return pl.pallas_call(
        _mid_body,
        grid=(GRID,),
        in_specs=[
            pl.BlockSpec((BR, 128), lambda i: (i, 0)),
            pl.BlockSpec((2, BR, 128), lambda i: (0, i, 0)),
            pl.BlockSpec((2, BR, 128), lambda i: (0, i, 0)),
            pl.BlockSpec((128, 256), lambda i: (0, 0)),
            pl.BlockSpec((128, 256), lambda i: (0, 0)),
            pl.BlockSpec((1, 256), lambda i: (0, 0)),
            pl.BlockSpec((256, 128), lambda i: (0, 0)),
            pl.BlockSpec((256, 128), lambda i: (0, 0)),
        ],
        out_specs=[
            pl.BlockSpec((BR, 128), lambda i: (i, 0)),
            pl.BlockSpec((BR, 128), lambda i: (i, 0)),
        ],
        out_shape=[
            jax.ShapeDtypeStruct((RN, 128), jnp.float32),
            jax.ShapeDtypeStruct((RN, 128), jnp.float32),
        ],
    )(h1p, p, cnt, Wl8, Wr8, b18, W2l8, W2r8)


def _fin_body(q_ref, cnt_ref, z_ref, b28_ref, w8_ref, b8_ref, o_ref):
    inv = 1.0 / jnp.maximum(cnt_ref[0] + cnt_ref[1], 1.0)
    a2 = (q_ref[0] + q_ref[1]) * inv
    h3 = jnp.maximum(a2 + z_ref[...] + b28_ref[...], 0.0)
    o = jnp.dot(h3, w8_ref[...], preferred_element_type=jnp.float32) + b8_ref[...]
    parts = []
    for k in range(PACK):
        ok = o[:, 3 * k:3 * k + 3]
        g = jax.nn.sigmoid(ok[:, 1:2])
        fsi = jnp.maximum(ok[:, 0:1], 0.0) + g
        mxi = jax.nn.sigmoid(ok[:, 2:3])
        parts.append(jnp.concatenate([fsi, g, mxi], axis=1))
    o_ref[...] = jnp.concatenate(parts, axis=1)


def _tc_fin(q, cnt, z, b28, W8, b8):
    return pl.pallas_call(
        _fin_body,
        grid=(GRID,),
        in_specs=[
            pl.BlockSpec((2, BR, 128), lambda i: (0, i, 0)),
            pl.BlockSpec((2, BR, 128), lambda i: (0, i, 0)),
            pl.BlockSpec((BR, 128), lambda i: (i, 0)),
            pl.BlockSpec((1, 128), lambda i: (0, 0)),
            pl.BlockSpec((128, 24), lambda i: (0, 0)),
            pl.BlockSpec((1, 24), lambda i: (0, 0)),
        ],
        out_specs=pl.BlockSpec((BR, 24), lambda i: (i, 0)),
        out_shape=jax.ShapeDtypeStruct((RN, 24), jnp.float32),
    )(q, cnt, z, b28, W8, b8)


def kernel(x, edge_index, fc1_W, fc1_b, c1_Wl, c1_Wr, c1_b,
           c2_Wl, c2_Wr, c2_b, fc2_W, fc2_b):
    eye8 = jnp.eye(PACK, dtype=jnp.float32)
    W1B = jnp.kron(eye8, fc1_W)
    b128 = jnp.tile(fc1_b, PACK).reshape(1, 128)
    Wl8 = jnp.kron(eye8, c1_Wl)
    Wr8 = jnp.kron(eye8, c1_Wr)
    W2l8 = jnp.kron(eye8, c2_Wl)
    W2r8 = jnp.kron(eye8, c2_Wr)
    W8 = jnp.kron(eye8, fc2_W)
    b18 = jnp.tile(c1_b, PACK).reshape(1, 256)
    b28 = jnp.tile(c2_b, PACK).reshape(1, 128)
    b8 = jnp.tile(fc2_b, PACK).reshape(1, 24)

    srce = edge_index[0]
    dste = edge_index[1]
    cnt = _sc_count()(dste).reshape(2, RN, 128)
    h1p = _tc_fc1(x.reshape(RN, PACK * 128), W1B, b128)
    p = _sc_agg()(h1p.reshape(N, D), srce, dste).reshape(2, RN, 128)
    y2p, z2p = _tc_mid(h1p, p, cnt, Wl8, Wr8, b18, W2l8, W2r8)
    q = _sc_agg()(y2p.reshape(N, D), srce, dste).reshape(2, RN, 128)
    out24 = _tc_fin(q, cnt, z2p, b28, W8, b8)
    return out24.reshape(N, 3)
